# baseline (speedup 1.0000x reference)
import functools

import jax
import jax.numpy as jnp
from jax import lax
from jax.experimental import pallas as pl
from jax.experimental.pallas import tpu as pltpu
from jax.experimental.pallas import tpu_sc as plsc

_f32 = jnp.float32
_i32 = jnp.int32

N = 10000
E = 320000
D = 128
C = 40

NW = 32
NPW = 320
NPAD = NW * NPW
SCH = 512
NSCH = E // SCH
GCH = 128
NGCH = E // GCH
RPS = 632
NACC = 16 * RPS


def _sc_mesh():
    return plsc.VectorSubcoreMesh(core_axis_name="c", subcore_axis_name="s")


def _onehot0():
    lane = lax.iota(_i32, 16)
    one = jnp.full((16,), 1.0, _f32)
    zero = jnp.full((16,), 0.0, _f32)
    return jnp.where(lane == jnp.zeros((16,), _i32), one, zero)


_SC_PARAMS = pltpu.CompilerParams(needs_layout_passes=False)


EPW = E // NW
BGRP = EPW // 16
PRV = 10304
ROWL = 10752
OMUL = 6554
OSH = 9


def _bin_body(srca, dsta, pairs, counts, offs, srcv, dstv, pairsv, hist,
              offsv, cursv):
    wid = lax.axis_index("c") * 16 + lax.axis_index("s")
    zi = jnp.zeros((16,), _i32)
    omulv = jnp.full((16,), OMUL, _i32)
    npwv = jnp.full((16,), NPW, _i32)
    shov = jnp.full((16,), 21, _i32)
    sh9 = jnp.full((16,), OSH, _i32)
    m511 = jnp.full((16,), 511, _i32)
    onev = jnp.full((16,), 1, _i32)

    pltpu.sync_copy(dsta.at[pl.ds(wid * EPW, EPW)], dstv)
    pltpu.sync_copy(srca.at[pl.ds(wid * EPW, EPW)], srcv)
    for k in range(3):
        hist[pl.ds(k * 16, 16)] = zi

    def h_g(g, _):
        d16 = dstv[pl.ds(g * 16, 16)]
        ow = (d16 * omulv) >> shov
        rank, last = plsc.scan_count(ow)
        plsc.addupdate_scatter(hist, [ow], rank, mask=last)
        return 0

    lax.fori_loop(0, BGRP, h_g, 0)

    sev = jnp.full((16,), 7, _i32)
    m8 = jnp.full((16,), ~7, _i32)
    h0 = hist[pl.ds(0, 16)]
    h1 = hist[pl.ds(16, 16)]
    hp0 = (h0 + sev) & m8
    hp1 = (h1 + sev) & m8
    c0 = plsc.cumsum(hp0)
    c1 = plsc.cumsum(hp1)
    off0 = c0 - hp0
    off1 = c1 - hp1 + jnp.full((16,), c0[15], _i32)
    offsv[pl.ds(0, 16)] = off0
    offsv[pl.ds(16, 16)] = off1
    offsv[pl.ds(32, 16)] = zi
    cursv[pl.ds(0, 16)] = off0
    cursv[pl.ds(16, 16)] = off1

    def p_g(g, _):
        d16 = dstv[pl.ds(g * 16, 16)]
        s16 = srcv[pl.ds(g * 16, 16)]
        ow = (d16 * omulv) >> shov
        dloc = d16 - ow * npwv
        packed = (s16 << sh9) | dloc
        rank, last = plsc.scan_count(ow)
        cur = plsc.load_gather(cursv, [ow])
        pos = cur + rank - onev
        plsc.store_scatter(pairsv, [pos], packed)
        plsc.addupdate_scatter(cursv, [ow], rank, mask=last)
        return 0

    lax.fori_loop(0, BGRP, p_g, 0)

    pltpu.sync_copy(pairsv, pairs.at[pl.ds(wid * ROWL, PRV)])
    pltpu.sync_copy(hist, counts.at[pl.ds(wid * 48, 48)])
    pltpu.sync_copy(offsv, offs.at[pl.ds(wid * 48, 48)])


def _bin_call(src, dst):
    fn = pl.kernel(
        _bin_body,
        out_type=(jax.ShapeDtypeStruct((NW * ROWL,), _i32),
                  jax.ShapeDtypeStruct((NW * 48,), _i32),
                  jax.ShapeDtypeStruct((NW * 48,), _i32)),
        mesh=_sc_mesh(),
        compiler_params=_SC_PARAMS,
        scratch_types=[
            pltpu.VMEM((EPW,), _i32),
            pltpu.VMEM((EPW,), _i32),
            pltpu.VMEM((PRV,), _i32),
            pltpu.VMEM((48,), _i32),
            pltpu.VMEM((48,), _i32),
            pltpu.VMEM((48,), _i32),
        ],
    )
    return fn(src, dst)


def _maxb_body(feat, pairs, counts, offs, out, deg, acc, acc1, degv, cntv,
               offv, pbuf, srcbuf, dlbuf, rowsr, semg):
    wid = lax.axis_index("c") * 16 + lax.axis_index("s")
    lo = wid * NPW
    ninf = jnp.full((16,), -jnp.inf, _f32)
    zf = jnp.zeros((16,), _f32)
    sh9 = jnp.full((16,), OSH, _i32)
    m511 = jnp.full((16,), 511, _i32)
    npwv = jnp.full((16,), NPW, _i32)
    lanev = lax.iota(_i32, 16)
    trash16 = (lanev << sh9) | npwv
    onehot = _onehot0()

    def init_row(r, _):
        for f in range(D // 16):
            acc[r, pl.ds(f * 16, 16)] = ninf
            acc1[r, pl.ds(f * 16, 16)] = ninf
        return 0

    lax.fori_loop(0, NPW + 16, init_row, 0)

    def init_deg(r, _):
        degv[pl.ds(r * 16, 16)] = zf
        return 0

    lax.fori_loop(0, (NPW + 32) // 16, init_deg, 0)

    def initb(q, _):
        pbuf[pl.ds(512 + q * 16, 16)] = trash16
        return 0

    lax.fori_loop(0, 2, initb, 0)

    pltpu.sync_copy(counts, cntv.at[pl.ds(0, NW * 48)])
    pltpu.sync_copy(offs, offv.at[pl.ds(0, NW * 48)])

    def binner(b, _):
        cnt = cntv[pl.ds(b * 48 + wid, 16)][0]
        off = offv[pl.ds(b * 48 + wid, 16)][0]
        nchk = (cnt + 511) // 512

        def chunk(t, _):
            o = pl.multiple_of(b * ROWL + off + t * 512, 8)
            pltpu.sync_copy(pairs.at[pl.ds(o, 512)],
                            pbuf.at[pl.ds(0, 512)])
            valid = jnp.minimum(cnt - t * 512, 512)
            pbuf[pl.ds(valid, 16)] = trash16
            ngrp = (valid + 15) // 16

            @pl.when(ngrp > 0)
            def _():
                def sfill(q, _):
                    sv = pbuf[pl.ds(q * 16, 16)] >> sh9
                    sv = jnp.minimum(jnp.maximum(sv, jnp.zeros((16,), _i32)),
                                     jnp.full((16,), N - 1, _i32))
                    srcbuf[pl.ds(q * 16, 16)] = sv
                    return 0

                lax.fori_loop(0, 33, sfill, 0)
                nblk = (ngrp + 7) // 8
                pltpu.async_copy(feat.at[srcbuf.at[pl.ds(0, 128)]],
                                 rowsr.at[0], semg)

                def blk(k, _):
                    b2 = lax.rem(k, 2)
                    pltpu.make_async_copy(feat.at[srcbuf.at[pl.ds(0, 128)]],
                                          rowsr.at[b2], semg).wait()

                    @pl.when(k + 1 < nblk)
                    def _():
                        pltpu.async_copy(
                            feat.at[srcbuf.at[pl.ds((k + 1) * 128, 128)]],
                            rowsr.at[1 - b2], semg)

                    gcount = jnp.minimum(ngrp - k * 8, 8)

                    def grp(gg, _):
                        g = k * 8 + gg
                        pg = pbuf[pl.ds(g * 16, 16)]
                        dl16 = pg & m511
                        dlbuf[pl.ds(0, 16)] = dl16
                        rankd, lastd = plsc.scan_count(dl16)
                        plsc.addupdate_scatter(degv, [dl16],
                                               rankd.astype(_f32), mask=lastd)

                        def edge(e, _):
                            for kk, a in ((0, acc), (1, acc1)):
                                ee = e * 2 + kk
                                r = dlbuf[pl.ds(ee, 16)][0]
                                wb = gg * 16 + ee
                                for f in range(D // 16):
                                    sl = pl.ds(f * 16, 16)
                                    a[r, sl] = jnp.maximum(a[r, sl],
                                                           rowsr[b2, wb, sl])
                            return 0

                        lax.fori_loop(0, 8, edge, 0)
                        return 0

                    lax.fori_loop(0, gcount, grp, 0)
                    return 0

                lax.fori_loop(0, nblk, blk, 0)

            return 0

        lax.fori_loop(0, nchk, chunk, 0)
        return 0

    lax.fori_loop(0, NW, binner, 0)

    def comb(r, _):
        for f in range(D // 16):
            sl = pl.ds(f * 16, 16)
            acc[r, sl] = jnp.maximum(acc[r, sl], acc1[r, sl])
        return 0

    lax.fori_loop(0, NPW, comb, 0)

    pltpu.sync_copy(acc.at[pl.ds(0, NPW)], out.at[pl.ds(lo, NPW)])
    pltpu.sync_copy(degv.at[pl.ds(0, NPW)], deg.at[pl.ds(lo, NPW)])


def _maxb_call(feat, pairs, counts, offs):
    fn = pl.kernel(
        _maxb_body,
        out_type=(jax.ShapeDtypeStruct((NPAD, D), _f32),
                  jax.ShapeDtypeStruct((NPAD,), _f32)),
        mesh=_sc_mesh(),
        compiler_params=_SC_PARAMS,
        scratch_types=[
            pltpu.VMEM((NPW + 16, D), _f32),
            pltpu.VMEM((NPW + 16, D), _f32),
            pltpu.VMEM((NPW + 32,), _f32),
            pltpu.VMEM((NW * 48 + 16,), _i32),
            pltpu.VMEM((NW * 48 + 16,), _i32),
            pltpu.VMEM((544,), _i32),
            pltpu.VMEM((544,), _i32),
            pltpu.VMEM((32,), _i32),
            pltpu.VMEM((2, 128, D), _f32),
            pltpu.SemaphoreType.DMA,
        ],
    )
    return fn(feat, pairs, counts, offs)


def _max_deg_body(feat, srca, dsta, out, deg, acc, degv, dstc2, srcc2, mdst,
                  msrc, rows2, semd, sems, semg):
    wid = lax.axis_index("c") * 16 + lax.axis_index("s")
    lo = wid * NPW
    ninf = jnp.full((16,), -jnp.inf, _f32)
    zf = jnp.zeros((16,), _f32)
    zi = jnp.zeros((16,), _i32)
    lov = jnp.full((16,), lo, _i32)
    npwv = jnp.full((16,), NPW, _i32)
    onehot = _onehot0()

    def init_row(r, _):
        for f in range(D // 16):
            acc[r, pl.ds(f * 16, 16)] = ninf
        return 0

    lax.fori_loop(0, NPW + 16, init_row, 0)

    def init_deg(r, _):
        degv[pl.ds(r * 16, 16)] = zf
        return 0

    lax.fori_loop(0, (NPW + 32) // 16, init_deg, 0)

    pltpu.async_copy(dsta.at[pl.ds(0, SCH)], dstc2.at[0], semd)
    pltpu.async_copy(srca.at[pl.ds(0, SCH)], srcc2.at[0], sems)

    def process(buf, next_ch):
        pltpu.make_async_copy(dsta.at[pl.ds(0, SCH)], dstc2.at[buf],
                              semd).wait()
        pltpu.make_async_copy(srca.at[pl.ds(0, SCH)], srcc2.at[buf],
                              sems).wait()

        @pl.when(next_ch < NSCH)
        def _():
            nbase = next_ch * SCH
            pltpu.async_copy(dsta.at[pl.ds(nbase, SCH)], dstc2.at[1 - buf],
                             semd)
            pltpu.async_copy(srca.at[pl.ds(nbase, SCH)], srcc2.at[1 - buf],
                             sems)

        def group(g, cnt):
            d16 = dstc2[buf, pl.ds(g * 16, 16)]
            dloc = d16 - lov
            m = (dloc >= zi) & (dloc < npwv)
            s16 = srcc2[buf, pl.ds(g * 16, 16)]
            plsc.store_compressed(mdst.at[pl.ds(cnt, 16)], dloc, mask=m)
            plsc.store_compressed(msrc.at[pl.ds(cnt, 16)], s16, mask=m)
            return cnt + plsc.all_reduce_population_count(m)[0]

        M = lax.fori_loop(0, SCH // 16, group, 0)
        mdst[pl.ds(M, 16)] = npwv
        msrc[pl.ds(M, 16)] = lax.iota(_i32, 16)
        ngrp = (M + 15) // 16

        @pl.when(ngrp > 0)
        def _():
            idx0 = msrc[pl.ds(0, 16)]
            pltpu.async_copy(feat.at[idx0], rows2.at[0], semg)

            def proc(g, _):
                b = lax.rem(g, 2)
                pltpu.make_async_copy(feat.at[idx0], rows2.at[b], semg).wait()

                @pl.when(g + 1 < ngrp)
                def _():
                    idxn = msrc[pl.ds((g + 1) * 16, 16)]
                    pltpu.async_copy(feat.at[idxn], rows2.at[1 - b], semg)

                def edge(e, _):
                    r = mdst[pl.ds(g * 16 + e, 16)][0]
                    for f in range(D // 16):
                        sl = pl.ds(f * 16, 16)
                        acc[r, sl] = jnp.maximum(acc[r, sl], rows2[b, e, sl])
                    dsl = pl.ds(r, 16)
                    degv[dsl] = degv[dsl] + onehot
                    return 0

                lax.fori_loop(0, 16, edge, 0)
                return 0

            lax.fori_loop(0, ngrp, proc, 0)

        return 0

    def pair(j, _):
        process(0, 2 * j + 1)
        process(1, 2 * j + 2)
        return 0

    lax.fori_loop(0, NSCH // 2, pair, 0)
    process(0, jnp.int32(NSCH))

    pltpu.sync_copy(acc.at[pl.ds(0, NPW)], out.at[pl.ds(lo, NPW)])
    pltpu.sync_copy(degv.at[pl.ds(0, NPW)], deg.at[pl.ds(lo, NPW)])


def _max_deg_call(feat, src, dst):
    fn = pl.kernel(
        _max_deg_body,
        out_type=(jax.ShapeDtypeStruct((NPAD, D), _f32),
                  jax.ShapeDtypeStruct((NPAD,), _f32)),
        mesh=_sc_mesh(),
        compiler_params=_SC_PARAMS,
        scratch_types=[
            pltpu.VMEM((NPW + 16, D), _f32),
            pltpu.VMEM((NPW + 32,), _f32),
            pltpu.VMEM((2, SCH), _i32),
            pltpu.VMEM((2, SCH), _i32),
            pltpu.VMEM((SCH + 32,), _i32),
            pltpu.VMEM((SCH + 32,), _i32),
            pltpu.VMEM((2, 16, D), _f32),
            pltpu.SemaphoreType.DMA,
            pltpu.SemaphoreType.DMA,
            pltpu.SemaphoreType.DMA,
        ],
    )
    return fn(feat, src, dst)


def _sum_body(h, src2, dst2, out, acc, srcc2, dstc2, rows2, semis, semid,
              semg, sema):
    c = lax.axis_index("c")
    s = lax.axis_index("s")
    wid = c * 16 + s
    zf = jnp.zeros((16,), _f32)
    cw = (NGCH - wid + NW - 1) // NW

    def zrow(r, _):
        for f in range(D // 16):
            rows2[0, r, pl.ds(f * 16, 16)] = zf
        return 0

    lax.fori_loop(0, GCH, zrow, 0)
    base = s * RPS

    def zacc(t, _):
        pltpu.sync_copy(rows2.at[0], acc.at[pl.ds(base + t * GCH, GCH)])
        return 0

    lax.fori_loop(0, RPS // GCH, zacc, 0)
    tail = RPS - (RPS // GCH) * GCH
    pltpu.sync_copy(rows2.at[0].at[pl.ds(0, tail)],
                    acc.at[pl.ds(base + (RPS // GCH) * GCH, tail)])
    plsc.subcore_barrier()

    def fire_idx(j, b):
        ch = wid + j * NW
        pltpu.async_copy(src2.at[ch], srcc2.at[b], semis)
        pltpu.async_copy(dst2.at[ch], dstc2.at[b], semid)

    def wait_idx(b):
        pltpu.make_async_copy(src2.at[0], srcc2.at[b], semis).wait()
        pltpu.make_async_copy(dst2.at[0], dstc2.at[b], semid).wait()

    def fire_gather(b):
        pltpu.async_copy(h.at[srcc2.at[b]], rows2.at[b], semg)

    def wait_gather(b):
        pltpu.make_async_copy(h.at[srcc2.at[0]], rows2.at[b], semg).wait()

    def fire_scatter(b):
        pltpu.async_copy(rows2.at[b], acc.at[dstc2.at[b]], sema, add=True)

    def wait_scatter():
        pltpu.make_async_copy(rows2.at[0], acc.at[dstc2.at[0]], sema).wait()

    @pl.when(cw > 0)
    def _():
        fire_idx(0, 0)
        wait_idx(0)

        @pl.when(cw > 1)
        def _():
            fire_idx(1, 1)

        fire_gather(0)

    def step(j, _):
        b = lax.rem(j, 2)
        nb = 1 - b
        wait_gather(b)
        fire_scatter(b)

        @pl.when(j + 1 < cw)
        def _():
            wait_idx(nb)

            @pl.when(j + 2 < cw)
            def _():
                fire_idx(j + 2, b)

            @pl.when(j >= 1)
            def _():
                wait_scatter()

            fire_gather(nb)

        return 0

    lax.fori_loop(0, cw, step, 0)

    @pl.when(cw >= 1)
    def _():
        wait_scatter()

    @pl.when(cw >= 2)
    def _():
        wait_scatter()

    plsc.subcore_barrier()
    pltpu.sync_copy(acc.at[pl.ds(s * RPS, RPS)],
                    out.at[c].at[pl.ds(s * RPS, RPS)])


def _sum_call(h, src2, dst2):
    fn = pl.kernel(
        _sum_body,
        out_type=jax.ShapeDtypeStruct((2, NACC, D), _f32),
        mesh=_sc_mesh(),
        compiler_params=_SC_PARAMS,
        scratch_types=[
            pltpu.VMEM_SHARED((NACC, D), _f32),
            pltpu.VMEM((2, GCH), _i32),
            pltpu.VMEM((2, GCH), _i32),
            pltpu.VMEM((2, GCH, D), _f32),
            pltpu.SemaphoreType.DMA,
            pltpu.SemaphoreType.DMA,
            pltpu.SemaphoreType.DMA,
            pltpu.SemaphoreType.DMA,
        ],
    )
    return fn(h, src2, dst2)


_BR = 1000


def _layer0_body(x_ref, a_ref, w_ref, b_ref, o_ref):
    a = a_ref[...]
    agg = jnp.where(jnp.isfinite(a), a, 0.0)
    rst = x_ref[...] + agg
    o_ref[...] = jnp.maximum(
        jnp.dot(rst, w_ref[...], preferred_element_type=_f32) + b_ref[...], 0.0)


def _layer1_body(x_ref, p0_ref, p1_ref, w_ref, b_ref, o_ref):
    rst = x_ref[...] + p0_ref[...] + p1_ref[...]
    o_ref[...] = jnp.maximum(
        jnp.dot(rst, w_ref[...], preferred_element_type=_f32) + b_ref[...], 0.0)


def _layer2_body(x_ref, p0_ref, p1_ref, d_ref, w_ref, b_ref, o_ref):
    dinv = 1.0 / jnp.maximum(d_ref[...], 1.0)
    rst = x_ref[...] + (p0_ref[...] + p1_ref[...]) * dinv
    o_ref[...] = jnp.dot(rst, w_ref[...], preferred_element_type=_f32) + b_ref[...]


def _row_spec(cols):
    return pl.BlockSpec((_BR, cols), lambda i: (i, 0))


def _full_spec(r, c):
    return pl.BlockSpec((r, c), lambda i: (0, 0))


def _layer0_call(x, a, w, b):
    return pl.pallas_call(
        _layer0_body,
        grid=(N // _BR,),
        in_specs=[_row_spec(D), _row_spec(D), _full_spec(D, D), _full_spec(1, D)],
        out_specs=_row_spec(D),
        out_shape=jax.ShapeDtypeStruct((N, D), _f32),
    )(x, a, w, b)


def _layer1_call(x, p0, p1, w, b):
    return pl.pallas_call(
        _layer1_body,
        grid=(N // _BR,),
        in_specs=[_row_spec(D), _row_spec(D), _row_spec(D), _full_spec(D, D),
                  _full_spec(1, D)],
        out_specs=_row_spec(D),
        out_shape=jax.ShapeDtypeStruct((N, D), _f32),
    )(x, p0, p1, w, b)


def _layer2_call(x, p0, p1, d, w, b):
    return pl.pallas_call(
        _layer2_body,
        grid=(N // _BR,),
        in_specs=[_row_spec(D), _row_spec(D), _row_spec(D), _row_spec(1),
                  _full_spec(D, C), _full_spec(1, C)],
        out_specs=_row_spec(C),
        out_shape=jax.ShapeDtypeStruct((N, C), _f32),
    )(x, p0, p1, d, w, b)


def kernel(features, edge_index, W0, b0, W1, b1, W2, b2):
    src = edge_index[0]
    dst = edge_index[1]
    pairs, counts, offs = _bin_call(src, dst)
    aggp, degp = _maxb_call(features, pairs, counts, offs)
    agg0 = aggp[:N]
    deg = degp[:N].reshape(N, 1)
    h1 = _layer0_call(features, agg0, W0, b0.reshape(1, D))
    src2 = src.reshape(NGCH, GCH)
    dst2 = dst.reshape(NGCH, GCH)
    p = _sum_call(h1, src2, dst2)
    h2 = _layer1_call(h1, p[0, :N], p[1, :N], W1, b1.reshape(1, D))
    p2 = _sum_call(h2, src2, dst2)
    return _layer2_call(h2, p2[0, :N], p2[1, :N], deg, W2, b2.reshape(1, C))


# speedup vs baseline: 1.1250x; 1.1250x over previous
import functools

import jax
import jax.numpy as jnp
from jax import lax
from jax.experimental import pallas as pl
from jax.experimental.pallas import tpu as pltpu
from jax.experimental.pallas import tpu_sc as plsc

_f32 = jnp.float32
_i32 = jnp.int32

N = 10000
E = 320000
D = 128
C = 40

NW = 32
NPW = 320
NPAD = NW * NPW
SCH = 512
NSCH = E // SCH
GCH = 128
NGCH = E // GCH
RPS = 632
NACC = 16 * RPS


def _sc_mesh():
    return plsc.VectorSubcoreMesh(core_axis_name="c", subcore_axis_name="s")


def _onehot0():
    lane = lax.iota(_i32, 16)
    one = jnp.full((16,), 1.0, _f32)
    zero = jnp.full((16,), 0.0, _f32)
    return jnp.where(lane == jnp.zeros((16,), _i32), one, zero)


_SC_PARAMS = pltpu.CompilerParams(needs_layout_passes=False)


EPW = E // NW
BGRP = EPW // 16
PRV = 10304
ROWL = 10752
OMUL = 6554
OSH = 9


def _bin_body(srca, dsta, pairs, counts, offs, srcv, dstv, pairsv, hist,
              offsv, cursv):
    wid = lax.axis_index("c") * 16 + lax.axis_index("s")
    zi = jnp.zeros((16,), _i32)
    omulv = jnp.full((16,), OMUL, _i32)
    npwv = jnp.full((16,), NPW, _i32)
    shov = jnp.full((16,), 21, _i32)
    sh9 = jnp.full((16,), OSH, _i32)
    m511 = jnp.full((16,), 511, _i32)
    onev = jnp.full((16,), 1, _i32)

    pltpu.sync_copy(dsta.at[pl.ds(wid * EPW, EPW)], dstv)
    pltpu.sync_copy(srca.at[pl.ds(wid * EPW, EPW)], srcv)
    for k in range(3):
        hist[pl.ds(k * 16, 16)] = zi

    def h_g(g, _):
        d16 = dstv[pl.ds(g * 16, 16)]
        ow = (d16 * omulv) >> shov
        rank, last = plsc.scan_count(ow)
        plsc.addupdate_scatter(hist, [ow], rank, mask=last)
        return 0

    lax.fori_loop(0, BGRP, h_g, 0)

    sev = jnp.full((16,), 7, _i32)
    m8 = jnp.full((16,), ~7, _i32)
    h0 = hist[pl.ds(0, 16)]
    h1 = hist[pl.ds(16, 16)]
    hp0 = (h0 + sev) & m8
    hp1 = (h1 + sev) & m8
    c0 = plsc.cumsum(hp0)
    c1 = plsc.cumsum(hp1)
    off0 = c0 - hp0
    off1 = c1 - hp1 + jnp.full((16,), c0[15], _i32)
    offsv[pl.ds(0, 16)] = off0
    offsv[pl.ds(16, 16)] = off1
    offsv[pl.ds(32, 16)] = zi
    cursv[pl.ds(0, 16)] = off0
    cursv[pl.ds(16, 16)] = off1

    def p_g(g, _):
        d16 = dstv[pl.ds(g * 16, 16)]
        s16 = srcv[pl.ds(g * 16, 16)]
        ow = (d16 * omulv) >> shov
        dloc = d16 - ow * npwv
        packed = (s16 << sh9) | dloc
        rank, last = plsc.scan_count(ow)
        cur = plsc.load_gather(cursv, [ow])
        pos = cur + rank - onev
        plsc.store_scatter(pairsv, [pos], packed)
        plsc.addupdate_scatter(cursv, [ow], rank, mask=last)
        return 0

    lax.fori_loop(0, BGRP, p_g, 0)

    pltpu.sync_copy(pairsv, pairs.at[pl.ds(wid * ROWL, PRV)])
    pltpu.sync_copy(hist, counts.at[pl.ds(wid * 48, 48)])
    pltpu.sync_copy(offsv, offs.at[pl.ds(wid * 48, 48)])


def _bin_call(src, dst):
    fn = pl.kernel(
        _bin_body,
        out_type=(jax.ShapeDtypeStruct((NW * ROWL,), _i32),
                  jax.ShapeDtypeStruct((NW * 48,), _i32),
                  jax.ShapeDtypeStruct((NW * 48,), _i32)),
        mesh=_sc_mesh(),
        compiler_params=_SC_PARAMS,
        scratch_types=[
            pltpu.VMEM((EPW,), _i32),
            pltpu.VMEM((EPW,), _i32),
            pltpu.VMEM((PRV,), _i32),
            pltpu.VMEM((48,), _i32),
            pltpu.VMEM((48,), _i32),
            pltpu.VMEM((48,), _i32),
        ],
    )
    return fn(src, dst)


def _maxb_body(feat, pairs, counts, offs, out, deg, acc, acc1, degv, cntv,
               offv, pbuf, srcbuf, dlbuf, rowsr, semg):
    wid = lax.axis_index("c") * 16 + lax.axis_index("s")
    lo = wid * NPW
    ninf = jnp.full((16,), -jnp.inf, _f32)
    zf = jnp.zeros((16,), _f32)
    sh9 = jnp.full((16,), OSH, _i32)
    m511 = jnp.full((16,), 511, _i32)
    npwv = jnp.full((16,), NPW, _i32)
    lanev = lax.iota(_i32, 16)
    trash16 = (lanev << sh9) | npwv
    onehot = _onehot0()

    def init_row(r, _):
        for f in range(D // 16):
            acc[r, pl.ds(f * 16, 16)] = ninf
            acc1[r, pl.ds(f * 16, 16)] = ninf
        return 0

    lax.fori_loop(0, NPW + 16, init_row, 0)

    def init_deg(r, _):
        degv[pl.ds(r * 16, 16)] = zf
        return 0

    lax.fori_loop(0, (NPW + 32) // 16, init_deg, 0)

    def initb(q, _):
        pbuf[pl.ds(512 + q * 16, 16)] = trash16
        return 0

    lax.fori_loop(0, 2, initb, 0)

    pltpu.sync_copy(counts, cntv.at[pl.ds(0, NW * 48)])
    pltpu.sync_copy(offs, offv.at[pl.ds(0, NW * 48)])

    def binner(b, _):
        cnt = cntv[pl.ds(b * 48 + wid, 16)][0]
        off = offv[pl.ds(b * 48 + wid, 16)][0]
        nchk = (cnt + 511) // 512

        def chunk(t, _):
            o = pl.multiple_of(b * ROWL + off + t * 512, 8)
            pltpu.sync_copy(pairs.at[pl.ds(o, 512)],
                            pbuf.at[pl.ds(0, 512)])
            valid = jnp.minimum(cnt - t * 512, 512)
            pbuf[pl.ds(valid, 16)] = trash16
            ngrp = (valid + 15) // 16

            @pl.when(ngrp > 0)
            def _():
                def sfill(q, _):
                    sv = pbuf[pl.ds(q * 16, 16)] >> sh9
                    sv = jnp.minimum(jnp.maximum(sv, jnp.zeros((16,), _i32)),
                                     jnp.full((16,), N - 1, _i32))
                    srcbuf[pl.ds(q * 16, 16)] = sv
                    return 0

                lax.fori_loop(0, 33, sfill, 0)
                nblk = (ngrp + 7) // 8
                pltpu.async_copy(feat.at[srcbuf.at[pl.ds(0, 128)]],
                                 rowsr.at[0], semg)

                def blk(k, _):
                    b2 = lax.rem(k, 2)
                    pltpu.make_async_copy(feat.at[srcbuf.at[pl.ds(0, 128)]],
                                          rowsr.at[b2], semg).wait()

                    @pl.when(k + 1 < nblk)
                    def _():
                        pltpu.async_copy(
                            feat.at[srcbuf.at[pl.ds((k + 1) * 128, 128)]],
                            rowsr.at[1 - b2], semg)

                    gcount = jnp.minimum(ngrp - k * 8, 8)

                    def grp(gg, _):
                        g = k * 8 + gg
                        pg = pbuf[pl.ds(g * 16, 16)]
                        dl16 = pg & m511
                        dlbuf[pl.ds(0, 16)] = dl16
                        rankd, lastd = plsc.scan_count(dl16)
                        plsc.addupdate_scatter(degv, [dl16],
                                               rankd.astype(_f32), mask=lastd)

                        def edge(e, _):
                            rs = []
                            for kk in range(4):
                                ee = e * 4 + kk
                                rs.append(dlbuf[pl.ds(ee, 16)][0])
                            for kk in range(4):
                                a = acc if kk % 2 == 0 else acc1
                                ee = e * 4 + kk
                                r = rs[kk]
                                wb = gg * 16 + ee
                                for f in range(D // 16):
                                    sl = pl.ds(f * 16, 16)
                                    a[r, sl] = jnp.maximum(a[r, sl],
                                                           rowsr[b2, wb, sl])
                            return 0

                        lax.fori_loop(0, 4, edge, 0)
                        return 0

                    lax.fori_loop(0, gcount, grp, 0)
                    return 0

                lax.fori_loop(0, nblk, blk, 0)

            return 0

        lax.fori_loop(0, nchk, chunk, 0)
        return 0

    lax.fori_loop(0, NW, binner, 0)

    def comb(r, _):
        for f in range(D // 16):
            sl = pl.ds(f * 16, 16)
            acc[r, sl] = jnp.maximum(acc[r, sl], acc1[r, sl])
        return 0

    lax.fori_loop(0, NPW, comb, 0)

    pltpu.sync_copy(acc.at[pl.ds(0, NPW)], out.at[pl.ds(lo, NPW)])
    pltpu.sync_copy(degv.at[pl.ds(0, NPW)], deg.at[pl.ds(lo, NPW)])


def _maxb_call(feat, pairs, counts, offs):
    fn = pl.kernel(
        _maxb_body,
        out_type=(jax.ShapeDtypeStruct((NPAD, D), _f32),
                  jax.ShapeDtypeStruct((NPAD,), _f32)),
        mesh=_sc_mesh(),
        compiler_params=_SC_PARAMS,
        scratch_types=[
            pltpu.VMEM((NPW + 16, D), _f32),
            pltpu.VMEM((NPW + 16, D), _f32),
            pltpu.VMEM((NPW + 32,), _f32),
            pltpu.VMEM((NW * 48 + 16,), _i32),
            pltpu.VMEM((NW * 48 + 16,), _i32),
            pltpu.VMEM((544,), _i32),
            pltpu.VMEM((544,), _i32),
            pltpu.VMEM((32,), _i32),
            pltpu.VMEM((2, 128, D), _f32),
            pltpu.SemaphoreType.DMA,
        ],
    )
    return fn(feat, pairs, counts, offs)


def _max_deg_body(feat, srca, dsta, out, deg, acc, degv, dstc2, srcc2, mdst,
                  msrc, rows2, semd, sems, semg):
    wid = lax.axis_index("c") * 16 + lax.axis_index("s")
    lo = wid * NPW
    ninf = jnp.full((16,), -jnp.inf, _f32)
    zf = jnp.zeros((16,), _f32)
    zi = jnp.zeros((16,), _i32)
    lov = jnp.full((16,), lo, _i32)
    npwv = jnp.full((16,), NPW, _i32)
    onehot = _onehot0()

    def init_row(r, _):
        for f in range(D // 16):
            acc[r, pl.ds(f * 16, 16)] = ninf
        return 0

    lax.fori_loop(0, NPW + 16, init_row, 0)

    def init_deg(r, _):
        degv[pl.ds(r * 16, 16)] = zf
        return 0

    lax.fori_loop(0, (NPW + 32) // 16, init_deg, 0)

    pltpu.async_copy(dsta.at[pl.ds(0, SCH)], dstc2.at[0], semd)
    pltpu.async_copy(srca.at[pl.ds(0, SCH)], srcc2.at[0], sems)

    def process(buf, next_ch):
        pltpu.make_async_copy(dsta.at[pl.ds(0, SCH)], dstc2.at[buf],
                              semd).wait()
        pltpu.make_async_copy(srca.at[pl.ds(0, SCH)], srcc2.at[buf],
                              sems).wait()

        @pl.when(next_ch < NSCH)
        def _():
            nbase = next_ch * SCH
            pltpu.async_copy(dsta.at[pl.ds(nbase, SCH)], dstc2.at[1 - buf],
                             semd)
            pltpu.async_copy(srca.at[pl.ds(nbase, SCH)], srcc2.at[1 - buf],
                             sems)

        def group(g, cnt):
            d16 = dstc2[buf, pl.ds(g * 16, 16)]
            dloc = d16 - lov
            m = (dloc >= zi) & (dloc < npwv)
            s16 = srcc2[buf, pl.ds(g * 16, 16)]
            plsc.store_compressed(mdst.at[pl.ds(cnt, 16)], dloc, mask=m)
            plsc.store_compressed(msrc.at[pl.ds(cnt, 16)], s16, mask=m)
            return cnt + plsc.all_reduce_population_count(m)[0]

        M = lax.fori_loop(0, SCH // 16, group, 0)
        mdst[pl.ds(M, 16)] = npwv
        msrc[pl.ds(M, 16)] = lax.iota(_i32, 16)
        ngrp = (M + 15) // 16

        @pl.when(ngrp > 0)
        def _():
            idx0 = msrc[pl.ds(0, 16)]
            pltpu.async_copy(feat.at[idx0], rows2.at[0], semg)

            def proc(g, _):
                b = lax.rem(g, 2)
                pltpu.make_async_copy(feat.at[idx0], rows2.at[b], semg).wait()

                @pl.when(g + 1 < ngrp)
                def _():
                    idxn = msrc[pl.ds((g + 1) * 16, 16)]
                    pltpu.async_copy(feat.at[idxn], rows2.at[1 - b], semg)

                def edge(e, _):
                    r = mdst[pl.ds(g * 16 + e, 16)][0]
                    for f in range(D // 16):
                        sl = pl.ds(f * 16, 16)
                        acc[r, sl] = jnp.maximum(acc[r, sl], rows2[b, e, sl])
                    dsl = pl.ds(r, 16)
                    degv[dsl] = degv[dsl] + onehot
                    return 0

                lax.fori_loop(0, 16, edge, 0)
                return 0

            lax.fori_loop(0, ngrp, proc, 0)

        return 0

    def pair(j, _):
        process(0, 2 * j + 1)
        process(1, 2 * j + 2)
        return 0

    lax.fori_loop(0, NSCH // 2, pair, 0)
    process(0, jnp.int32(NSCH))

    pltpu.sync_copy(acc.at[pl.ds(0, NPW)], out.at[pl.ds(lo, NPW)])
    pltpu.sync_copy(degv.at[pl.ds(0, NPW)], deg.at[pl.ds(lo, NPW)])


def _max_deg_call(feat, src, dst):
    fn = pl.kernel(
        _max_deg_body,
        out_type=(jax.ShapeDtypeStruct((NPAD, D), _f32),
                  jax.ShapeDtypeStruct((NPAD,), _f32)),
        mesh=_sc_mesh(),
        compiler_params=_SC_PARAMS,
        scratch_types=[
            pltpu.VMEM((NPW + 16, D), _f32),
            pltpu.VMEM((NPW + 32,), _f32),
            pltpu.VMEM((2, SCH), _i32),
            pltpu.VMEM((2, SCH), _i32),
            pltpu.VMEM((SCH + 32,), _i32),
            pltpu.VMEM((SCH + 32,), _i32),
            pltpu.VMEM((2, 16, D), _f32),
            pltpu.SemaphoreType.DMA,
            pltpu.SemaphoreType.DMA,
            pltpu.SemaphoreType.DMA,
        ],
    )
    return fn(feat, src, dst)


def _sum_body(h, src2, dst2, out, acc, srcc2, dstc2, rows2, semis, semid,
              semg, sema):
    c = lax.axis_index("c")
    s = lax.axis_index("s")
    wid = c * 16 + s
    zf = jnp.zeros((16,), _f32)
    cw = (NGCH - wid + NW - 1) // NW

    def zrow(r, _):
        for f in range(D // 16):
            rows2[0, r, pl.ds(f * 16, 16)] = zf
        return 0

    lax.fori_loop(0, GCH, zrow, 0)
    base = s * RPS

    def zacc(t, _):
        pltpu.sync_copy(rows2.at[0], acc.at[pl.ds(base + t * GCH, GCH)])
        return 0

    lax.fori_loop(0, RPS // GCH, zacc, 0)
    tail = RPS - (RPS // GCH) * GCH
    pltpu.sync_copy(rows2.at[0].at[pl.ds(0, tail)],
                    acc.at[pl.ds(base + (RPS // GCH) * GCH, tail)])
    plsc.subcore_barrier()

    def fire_idx(j, b):
        ch = wid + j * NW
        pltpu.async_copy(src2.at[ch], srcc2.at[b], semis)
        pltpu.async_copy(dst2.at[ch], dstc2.at[b], semid)

    def wait_idx(b):
        pltpu.make_async_copy(src2.at[0], srcc2.at[b], semis).wait()
        pltpu.make_async_copy(dst2.at[0], dstc2.at[b], semid).wait()

    def fire_gather(b):
        pltpu.async_copy(h.at[srcc2.at[b]], rows2.at[b], semg)

    def wait_gather(b):
        pltpu.make_async_copy(h.at[srcc2.at[0]], rows2.at[b], semg).wait()

    def fire_scatter(b):
        pltpu.async_copy(rows2.at[b], acc.at[dstc2.at[b]], sema, add=True)

    def wait_scatter():
        pltpu.make_async_copy(rows2.at[0], acc.at[dstc2.at[0]], sema).wait()

    @pl.when(cw > 0)
    def _():
        fire_idx(0, 0)
        wait_idx(0)

        @pl.when(cw > 1)
        def _():
            fire_idx(1, 1)

        fire_gather(0)

    def step(j, _):
        b = lax.rem(j, 2)
        nb = 1 - b
        wait_gather(b)
        fire_scatter(b)

        @pl.when(j + 1 < cw)
        def _():
            wait_idx(nb)

            @pl.when(j + 2 < cw)
            def _():
                fire_idx(j + 2, b)

            @pl.when(j >= 1)
            def _():
                wait_scatter()

            fire_gather(nb)

        return 0

    lax.fori_loop(0, cw, step, 0)

    @pl.when(cw >= 1)
    def _():
        wait_scatter()

    @pl.when(cw >= 2)
    def _():
        wait_scatter()

    plsc.subcore_barrier()
    pltpu.sync_copy(acc.at[pl.ds(s * RPS, RPS)],
                    out.at[c].at[pl.ds(s * RPS, RPS)])


def _sum_call(h, src2, dst2):
    fn = pl.kernel(
        _sum_body,
        out_type=jax.ShapeDtypeStruct((2, NACC, D), _f32),
        mesh=_sc_mesh(),
        compiler_params=_SC_PARAMS,
        scratch_types=[
            pltpu.VMEM_SHARED((NACC, D), _f32),
            pltpu.VMEM((2, GCH), _i32),
            pltpu.VMEM((2, GCH), _i32),
            pltpu.VMEM((2, GCH, D), _f32),
            pltpu.SemaphoreType.DMA,
            pltpu.SemaphoreType.DMA,
            pltpu.SemaphoreType.DMA,
            pltpu.SemaphoreType.DMA,
        ],
    )
    return fn(h, src2, dst2)


_BR = 1000


def _layer0_body(x_ref, a_ref, w_ref, b_ref, o_ref):
    a = a_ref[...]
    agg = jnp.where(jnp.isfinite(a), a, 0.0)
    rst = x_ref[...] + agg
    o_ref[...] = jnp.maximum(
        jnp.dot(rst, w_ref[...], preferred_element_type=_f32) + b_ref[...], 0.0)


def _layer1_body(x_ref, p0_ref, p1_ref, w_ref, b_ref, o_ref):
    rst = x_ref[...] + p0_ref[...] + p1_ref[...]
    o_ref[...] = jnp.maximum(
        jnp.dot(rst, w_ref[...], preferred_element_type=_f32) + b_ref[...], 0.0)


def _layer2_body(x_ref, p0_ref, p1_ref, d_ref, w_ref, b_ref, o_ref):
    dinv = 1.0 / jnp.maximum(d_ref[...], 1.0)
    rst = x_ref[...] + (p0_ref[...] + p1_ref[...]) * dinv
    o_ref[...] = jnp.dot(rst, w_ref[...], preferred_element_type=_f32) + b_ref[...]


def _row_spec(cols):
    return pl.BlockSpec((_BR, cols), lambda i: (i, 0))


def _full_spec(r, c):
    return pl.BlockSpec((r, c), lambda i: (0, 0))


def _layer0_call(x, a, w, b):
    return pl.pallas_call(
        _layer0_body,
        grid=(N // _BR,),
        in_specs=[_row_spec(D), _row_spec(D), _full_spec(D, D), _full_spec(1, D)],
        out_specs=_row_spec(D),
        out_shape=jax.ShapeDtypeStruct((N, D), _f32),
    )(x, a, w, b)


def _layer1_call(x, p0, p1, w, b):
    return pl.pallas_call(
        _layer1_body,
        grid=(N // _BR,),
        in_specs=[_row_spec(D), _row_spec(D), _row_spec(D), _full_spec(D, D),
                  _full_spec(1, D)],
        out_specs=_row_spec(D),
        out_shape=jax.ShapeDtypeStruct((N, D), _f32),
    )(x, p0, p1, w, b)


def _layer2_call(x, p0, p1, d, w, b):
    return pl.pallas_call(
        _layer2_body,
        grid=(N // _BR,),
        in_specs=[_row_spec(D), _row_spec(D), _row_spec(D), _row_spec(1),
                  _full_spec(D, C), _full_spec(1, C)],
        out_specs=_row_spec(C),
        out_shape=jax.ShapeDtypeStruct((N, C), _f32),
    )(x, p0, p1, d, w, b)


def kernel(features, edge_index, W0, b0, W1, b1, W2, b2):
    src = edge_index[0]
    dst = edge_index[1]
    pairs, counts, offs = _bin_call(src, dst)
    aggp, degp = _maxb_call(features, pairs, counts, offs)
    agg0 = aggp[:N]
    deg = degp[:N].reshape(N, 1)
    h1 = _layer0_call(features, agg0, W0, b0.reshape(1, D))
    src2 = src.reshape(NGCH, GCH)
    dst2 = dst.reshape(NGCH, GCH)
    p = _sum_call(h1, src2, dst2)
    h2 = _layer1_call(h1, p[0, :N], p[1, :N], W1, b1.reshape(1, D))
    p2 = _sum_call(h2, src2, dst2)
    return _layer2_call(h2, p2[0, :N], p2[1, :N], deg, W2, b2.reshape(1, C))


# speedup vs baseline: 1.1570x; 1.0285x over previous
import functools

import jax
import jax.numpy as jnp
from jax import lax
from jax.experimental import pallas as pl
from jax.experimental.pallas import tpu as pltpu
from jax.experimental.pallas import tpu_sc as plsc

_f32 = jnp.float32
_i32 = jnp.int32

N = 10000
E = 320000
D = 128
C = 40

NW = 32
NPW = 320
NPAD = NW * NPW
SCH = 512
NSCH = E // SCH
GCH = 128
NGCH = E // GCH
RPS = 632
NACC = 16 * RPS


def _sc_mesh():
    return plsc.VectorSubcoreMesh(core_axis_name="c", subcore_axis_name="s")


def _onehot0():
    lane = lax.iota(_i32, 16)
    one = jnp.full((16,), 1.0, _f32)
    zero = jnp.full((16,), 0.0, _f32)
    return jnp.where(lane == jnp.zeros((16,), _i32), one, zero)


_SC_PARAMS = pltpu.CompilerParams(needs_layout_passes=False)


EPW = E // NW
BGRP = EPW // 16
PRV = 10304
ROWL = 10752
OMUL = 6554
OSH = 9


def _bin_body(srca, dsta, pairs, counts, offs, srcv, dstv, pairsv, hist,
              offsv, cursv):
    wid = lax.axis_index("c") * 16 + lax.axis_index("s")
    zi = jnp.zeros((16,), _i32)
    omulv = jnp.full((16,), OMUL, _i32)
    npwv = jnp.full((16,), NPW, _i32)
    shov = jnp.full((16,), 21, _i32)
    sh9 = jnp.full((16,), OSH, _i32)
    m511 = jnp.full((16,), 511, _i32)
    onev = jnp.full((16,), 1, _i32)

    pltpu.sync_copy(dsta.at[pl.ds(wid * EPW, EPW)], dstv)
    pltpu.sync_copy(srca.at[pl.ds(wid * EPW, EPW)], srcv)
    for k in range(3):
        hist[pl.ds(k * 16, 16)] = zi

    def h_g(g, _):
        d16 = dstv[pl.ds(g * 16, 16)]
        ow = (d16 * omulv) >> shov
        rank, last = plsc.scan_count(ow)
        plsc.addupdate_scatter(hist, [ow], rank, mask=last)
        return 0

    lax.fori_loop(0, BGRP, h_g, 0)

    sev = jnp.full((16,), 7, _i32)
    m8 = jnp.full((16,), ~7, _i32)
    h0 = hist[pl.ds(0, 16)]
    h1 = hist[pl.ds(16, 16)]
    hp0 = (h0 + sev) & m8
    hp1 = (h1 + sev) & m8
    c0 = plsc.cumsum(hp0)
    c1 = plsc.cumsum(hp1)
    off0 = c0 - hp0
    off1 = c1 - hp1 + jnp.full((16,), c0[15], _i32)
    offsv[pl.ds(0, 16)] = off0
    offsv[pl.ds(16, 16)] = off1
    offsv[pl.ds(32, 16)] = zi
    cursv[pl.ds(0, 16)] = off0
    cursv[pl.ds(16, 16)] = off1

    def p_g(g, _):
        d16 = dstv[pl.ds(g * 16, 16)]
        s16 = srcv[pl.ds(g * 16, 16)]
        ow = (d16 * omulv) >> shov
        dloc = d16 - ow * npwv
        packed = (s16 << sh9) | dloc
        rank, last = plsc.scan_count(ow)
        cur = plsc.load_gather(cursv, [ow])
        pos = cur + rank - onev
        plsc.store_scatter(pairsv, [pos], packed)
        plsc.addupdate_scatter(cursv, [ow], rank, mask=last)
        return 0

    lax.fori_loop(0, BGRP, p_g, 0)

    pltpu.sync_copy(pairsv, pairs.at[pl.ds(wid * ROWL, PRV)])
    pltpu.sync_copy(hist, counts.at[pl.ds(wid * 48, 48)])
    pltpu.sync_copy(offsv, offs.at[pl.ds(wid * 48, 48)])


def _bin_call(src, dst):
    fn = pl.kernel(
        _bin_body,
        out_type=(jax.ShapeDtypeStruct((NW * ROWL,), _i32),
                  jax.ShapeDtypeStruct((NW * 48,), _i32),
                  jax.ShapeDtypeStruct((NW * 48,), _i32)),
        mesh=_sc_mesh(),
        compiler_params=_SC_PARAMS,
        scratch_types=[
            pltpu.VMEM((EPW,), _i32),
            pltpu.VMEM((EPW,), _i32),
            pltpu.VMEM((PRV,), _i32),
            pltpu.VMEM((48,), _i32),
            pltpu.VMEM((48,), _i32),
            pltpu.VMEM((48,), _i32),
        ],
    )
    return fn(src, dst)


def _maxb_body(feat, pairs, counts, offs, out, deg, acc, acc1, degv, cntv,
               offv, pbuf, srcbuf, dlbuf, rowsr, semg):
    wid = lax.axis_index("c") * 16 + lax.axis_index("s")
    lo = wid * NPW
    ninf = jnp.full((16,), -jnp.inf, _f32)
    zf = jnp.zeros((16,), _f32)
    sh9 = jnp.full((16,), OSH, _i32)
    m511 = jnp.full((16,), 511, _i32)
    npwv = jnp.full((16,), NPW, _i32)
    lanev = lax.iota(_i32, 16)
    trash16 = (lanev << sh9) | npwv
    onehot = _onehot0()

    def init_row(r, _):
        for f in range(D // 16):
            acc[r, pl.ds(f * 16, 16)] = ninf
            acc1[r, pl.ds(f * 16, 16)] = ninf
        return 0

    lax.fori_loop(0, NPW + 16, init_row, 0)

    def init_deg(r, _):
        degv[pl.ds(r * 16, 16)] = zf
        return 0

    lax.fori_loop(0, (NPW + 32) // 16, init_deg, 0)

    def initb(q, _):
        pbuf[pl.ds(512 + q * 16, 16)] = trash16
        return 0

    lax.fori_loop(0, 2, initb, 0)

    pltpu.sync_copy(counts, cntv.at[pl.ds(0, NW * 48)])
    pltpu.sync_copy(offs, offv.at[pl.ds(0, NW * 48)])

    def binner(b, _):
        cnt = cntv[pl.ds(b * 48 + wid, 16)][0]
        off = offv[pl.ds(b * 48 + wid, 16)][0]
        nchk = (cnt + 511) // 512

        def chunk(t, _):
            o = pl.multiple_of(b * ROWL + off + t * 512, 8)
            pltpu.sync_copy(pairs.at[pl.ds(o, 512)],
                            pbuf.at[pl.ds(0, 512)])
            valid = jnp.minimum(cnt - t * 512, 512)
            pbuf[pl.ds(valid, 16)] = trash16
            ngrp = (valid + 15) // 16

            @pl.when(ngrp > 0)
            def _():
                def sfill(q, _):
                    sv = pbuf[pl.ds(q * 16, 16)] >> sh9
                    sv = jnp.minimum(jnp.maximum(sv, jnp.zeros((16,), _i32)),
                                     jnp.full((16,), N - 1, _i32))
                    srcbuf[pl.ds(q * 16, 16)] = sv
                    return 0

                lax.fori_loop(0, 33, sfill, 0)
                nblk = (ngrp + 7) // 8
                pltpu.async_copy(feat.at[srcbuf.at[pl.ds(0, 128)]],
                                 rowsr.at[0], semg)

                def blk(k, _):
                    b2 = lax.rem(k, 2)
                    pltpu.make_async_copy(feat.at[srcbuf.at[pl.ds(0, 128)]],
                                          rowsr.at[b2], semg).wait()

                    @pl.when(k + 1 < nblk)
                    def _():
                        pltpu.async_copy(
                            feat.at[srcbuf.at[pl.ds((k + 1) * 128, 128)]],
                            rowsr.at[1 - b2], semg)

                    gcount = jnp.minimum(ngrp - k * 8, 8)

                    def grp(gg, _):
                        g = k * 8 + gg
                        pg = pbuf[pl.ds(g * 16, 16)]
                        dl16 = pg & m511
                        dlbuf[pl.ds(0, 16)] = dl16
                        rankd, lastd = plsc.scan_count(dl16)
                        plsc.addupdate_scatter(degv, [dl16],
                                               rankd.astype(_f32), mask=lastd)

                        rs = []
                        for ee in range(16):
                            rs.append(dlbuf[pl.ds(ee, 16)][0])
                        for ee in range(16):
                            a = acc if ee % 2 == 0 else acc1
                            r = rs[ee]
                            wb = gg * 16 + ee
                            for f in range(D // 16):
                                sl = pl.ds(f * 16, 16)
                                a[r, sl] = jnp.maximum(a[r, sl],
                                                       rowsr[b2, wb, sl])
                        return 0

                    lax.fori_loop(0, gcount, grp, 0)
                    return 0

                lax.fori_loop(0, nblk, blk, 0)

            return 0

        lax.fori_loop(0, nchk, chunk, 0)
        return 0

    lax.fori_loop(0, NW, binner, 0)

    def comb(r, _):
        for f in range(D // 16):
            sl = pl.ds(f * 16, 16)
            acc[r, sl] = jnp.maximum(acc[r, sl], acc1[r, sl])
        return 0

    lax.fori_loop(0, NPW, comb, 0)

    pltpu.sync_copy(acc.at[pl.ds(0, NPW)], out.at[pl.ds(lo, NPW)])
    pltpu.sync_copy(degv.at[pl.ds(0, NPW)], deg.at[pl.ds(lo, NPW)])


def _maxb_call(feat, pairs, counts, offs):
    fn = pl.kernel(
        _maxb_body,
        out_type=(jax.ShapeDtypeStruct((NPAD, D), _f32),
                  jax.ShapeDtypeStruct((NPAD,), _f32)),
        mesh=_sc_mesh(),
        compiler_params=_SC_PARAMS,
        scratch_types=[
            pltpu.VMEM((NPW + 16, D), _f32),
            pltpu.VMEM((NPW + 16, D), _f32),
            pltpu.VMEM((NPW + 32,), _f32),
            pltpu.VMEM((NW * 48 + 16,), _i32),
            pltpu.VMEM((NW * 48 + 16,), _i32),
            pltpu.VMEM((544,), _i32),
            pltpu.VMEM((544,), _i32),
            pltpu.VMEM((32,), _i32),
            pltpu.VMEM((2, 128, D), _f32),
            pltpu.SemaphoreType.DMA,
        ],
    )
    return fn(feat, pairs, counts, offs)


def _max_deg_body(feat, srca, dsta, out, deg, acc, degv, dstc2, srcc2, mdst,
                  msrc, rows2, semd, sems, semg):
    wid = lax.axis_index("c") * 16 + lax.axis_index("s")
    lo = wid * NPW
    ninf = jnp.full((16,), -jnp.inf, _f32)
    zf = jnp.zeros((16,), _f32)
    zi = jnp.zeros((16,), _i32)
    lov = jnp.full((16,), lo, _i32)
    npwv = jnp.full((16,), NPW, _i32)
    onehot = _onehot0()

    def init_row(r, _):
        for f in range(D // 16):
            acc[r, pl.ds(f * 16, 16)] = ninf
        return 0

    lax.fori_loop(0, NPW + 16, init_row, 0)

    def init_deg(r, _):
        degv[pl.ds(r * 16, 16)] = zf
        return 0

    lax.fori_loop(0, (NPW + 32) // 16, init_deg, 0)

    pltpu.async_copy(dsta.at[pl.ds(0, SCH)], dstc2.at[0], semd)
    pltpu.async_copy(srca.at[pl.ds(0, SCH)], srcc2.at[0], sems)

    def process(buf, next_ch):
        pltpu.make_async_copy(dsta.at[pl.ds(0, SCH)], dstc2.at[buf],
                              semd).wait()
        pltpu.make_async_copy(srca.at[pl.ds(0, SCH)], srcc2.at[buf],
                              sems).wait()

        @pl.when(next_ch < NSCH)
        def _():
            nbase = next_ch * SCH
            pltpu.async_copy(dsta.at[pl.ds(nbase, SCH)], dstc2.at[1 - buf],
                             semd)
            pltpu.async_copy(srca.at[pl.ds(nbase, SCH)], srcc2.at[1 - buf],
                             sems)

        def group(g, cnt):
            d16 = dstc2[buf, pl.ds(g * 16, 16)]
            dloc = d16 - lov
            m = (dloc >= zi) & (dloc < npwv)
            s16 = srcc2[buf, pl.ds(g * 16, 16)]
            plsc.store_compressed(mdst.at[pl.ds(cnt, 16)], dloc, mask=m)
            plsc.store_compressed(msrc.at[pl.ds(cnt, 16)], s16, mask=m)
            return cnt + plsc.all_reduce_population_count(m)[0]

        M = lax.fori_loop(0, SCH // 16, group, 0)
        mdst[pl.ds(M, 16)] = npwv
        msrc[pl.ds(M, 16)] = lax.iota(_i32, 16)
        ngrp = (M + 15) // 16

        @pl.when(ngrp > 0)
        def _():
            idx0 = msrc[pl.ds(0, 16)]
            pltpu.async_copy(feat.at[idx0], rows2.at[0], semg)

            def proc(g, _):
                b = lax.rem(g, 2)
                pltpu.make_async_copy(feat.at[idx0], rows2.at[b], semg).wait()

                @pl.when(g + 1 < ngrp)
                def _():
                    idxn = msrc[pl.ds((g + 1) * 16, 16)]
                    pltpu.async_copy(feat.at[idxn], rows2.at[1 - b], semg)

                def edge(e, _):
                    r = mdst[pl.ds(g * 16 + e, 16)][0]
                    for f in range(D // 16):
                        sl = pl.ds(f * 16, 16)
                        acc[r, sl] = jnp.maximum(acc[r, sl], rows2[b, e, sl])
                    dsl = pl.ds(r, 16)
                    degv[dsl] = degv[dsl] + onehot
                    return 0

                lax.fori_loop(0, 16, edge, 0)
                return 0

            lax.fori_loop(0, ngrp, proc, 0)

        return 0

    def pair(j, _):
        process(0, 2 * j + 1)
        process(1, 2 * j + 2)
        return 0

    lax.fori_loop(0, NSCH // 2, pair, 0)
    process(0, jnp.int32(NSCH))

    pltpu.sync_copy(acc.at[pl.ds(0, NPW)], out.at[pl.ds(lo, NPW)])
    pltpu.sync_copy(degv.at[pl.ds(0, NPW)], deg.at[pl.ds(lo, NPW)])


def _max_deg_call(feat, src, dst):
    fn = pl.kernel(
        _max_deg_body,
        out_type=(jax.ShapeDtypeStruct((NPAD, D), _f32),
                  jax.ShapeDtypeStruct((NPAD,), _f32)),
        mesh=_sc_mesh(),
        compiler_params=_SC_PARAMS,
        scratch_types=[
            pltpu.VMEM((NPW + 16, D), _f32),
            pltpu.VMEM((NPW + 32,), _f32),
            pltpu.VMEM((2, SCH), _i32),
            pltpu.VMEM((2, SCH), _i32),
            pltpu.VMEM((SCH + 32,), _i32),
            pltpu.VMEM((SCH + 32,), _i32),
            pltpu.VMEM((2, 16, D), _f32),
            pltpu.SemaphoreType.DMA,
            pltpu.SemaphoreType.DMA,
            pltpu.SemaphoreType.DMA,
        ],
    )
    return fn(feat, src, dst)


def _sum_body(h, src2, dst2, out, acc, srcc2, dstc2, rows2, semis, semid,
              semg, sema):
    c = lax.axis_index("c")
    s = lax.axis_index("s")
    wid = c * 16 + s
    zf = jnp.zeros((16,), _f32)
    cw = (NGCH - wid + NW - 1) // NW

    def zrow(r, _):
        for f in range(D // 16):
            rows2[0, r, pl.ds(f * 16, 16)] = zf
        return 0

    lax.fori_loop(0, GCH, zrow, 0)
    base = s * RPS

    def zacc(t, _):
        pltpu.sync_copy(rows2.at[0], acc.at[pl.ds(base + t * GCH, GCH)])
        return 0

    lax.fori_loop(0, RPS // GCH, zacc, 0)
    tail = RPS - (RPS // GCH) * GCH
    pltpu.sync_copy(rows2.at[0].at[pl.ds(0, tail)],
                    acc.at[pl.ds(base + (RPS // GCH) * GCH, tail)])
    plsc.subcore_barrier()

    def fire_idx(j, b):
        ch = wid + j * NW
        pltpu.async_copy(src2.at[ch], srcc2.at[b], semis)
        pltpu.async_copy(dst2.at[ch], dstc2.at[b], semid)

    def wait_idx(b):
        pltpu.make_async_copy(src2.at[0], srcc2.at[b], semis).wait()
        pltpu.make_async_copy(dst2.at[0], dstc2.at[b], semid).wait()

    def fire_gather(b):
        pltpu.async_copy(h.at[srcc2.at[b]], rows2.at[b], semg)

    def wait_gather(b):
        pltpu.make_async_copy(h.at[srcc2.at[0]], rows2.at[b], semg).wait()

    def fire_scatter(b):
        pltpu.async_copy(rows2.at[b], acc.at[dstc2.at[b]], sema, add=True)

    def wait_scatter():
        pltpu.make_async_copy(rows2.at[0], acc.at[dstc2.at[0]], sema).wait()

    @pl.when(cw > 0)
    def _():
        fire_idx(0, 0)
        wait_idx(0)

        @pl.when(cw > 1)
        def _():
            fire_idx(1, 1)

        fire_gather(0)

    def step(j, _):
        b = lax.rem(j, 2)
        nb = 1 - b
        wait_gather(b)
        fire_scatter(b)

        @pl.when(j + 1 < cw)
        def _():
            wait_idx(nb)

            @pl.when(j + 2 < cw)
            def _():
                fire_idx(j + 2, b)

            @pl.when(j >= 1)
            def _():
                wait_scatter()

            fire_gather(nb)

        return 0

    lax.fori_loop(0, cw, step, 0)

    @pl.when(cw >= 1)
    def _():
        wait_scatter()

    @pl.when(cw >= 2)
    def _():
        wait_scatter()

    plsc.subcore_barrier()
    pltpu.sync_copy(acc.at[pl.ds(s * RPS, RPS)],
                    out.at[c].at[pl.ds(s * RPS, RPS)])


def _sum_call(h, src2, dst2):
    fn = pl.kernel(
        _sum_body,
        out_type=jax.ShapeDtypeStruct((2, NACC, D), _f32),
        mesh=_sc_mesh(),
        compiler_params=_SC_PARAMS,
        scratch_types=[
            pltpu.VMEM_SHARED((NACC, D), _f32),
            pltpu.VMEM((2, GCH), _i32),
            pltpu.VMEM((2, GCH), _i32),
            pltpu.VMEM((2, GCH, D), _f32),
            pltpu.SemaphoreType.DMA,
            pltpu.SemaphoreType.DMA,
            pltpu.SemaphoreType.DMA,
            pltpu.SemaphoreType.DMA,
        ],
    )
    return fn(h, src2, dst2)


_BR = 1000


def _layer0_body(x_ref, a_ref, w_ref, b_ref, o_ref):
    a = a_ref[...]
    agg = jnp.where(jnp.isfinite(a), a, 0.0)
    rst = x_ref[...] + agg
    o_ref[...] = jnp.maximum(
        jnp.dot(rst, w_ref[...], preferred_element_type=_f32) + b_ref[...], 0.0)


def _layer1_body(x_ref, p0_ref, p1_ref, w_ref, b_ref, o_ref):
    rst = x_ref[...] + p0_ref[...] + p1_ref[...]
    o_ref[...] = jnp.maximum(
        jnp.dot(rst, w_ref[...], preferred_element_type=_f32) + b_ref[...], 0.0)


def _layer2_body(x_ref, p0_ref, p1_ref, d_ref, w_ref, b_ref, o_ref):
    dinv = 1.0 / jnp.maximum(d_ref[...], 1.0)
    rst = x_ref[...] + (p0_ref[...] + p1_ref[...]) * dinv
    o_ref[...] = jnp.dot(rst, w_ref[...], preferred_element_type=_f32) + b_ref[...]


def _row_spec(cols):
    return pl.BlockSpec((_BR, cols), lambda i: (i, 0))


def _full_spec(r, c):
    return pl.BlockSpec((r, c), lambda i: (0, 0))


def _layer0_call(x, a, w, b):
    return pl.pallas_call(
        _layer0_body,
        grid=(N // _BR,),
        in_specs=[_row_spec(D), _row_spec(D), _full_spec(D, D), _full_spec(1, D)],
        out_specs=_row_spec(D),
        out_shape=jax.ShapeDtypeStruct((N, D), _f32),
    )(x, a, w, b)


def _layer1_call(x, p0, p1, w, b):
    return pl.pallas_call(
        _layer1_body,
        grid=(N // _BR,),
        in_specs=[_row_spec(D), _row_spec(D), _row_spec(D), _full_spec(D, D),
                  _full_spec(1, D)],
        out_specs=_row_spec(D),
        out_shape=jax.ShapeDtypeStruct((N, D), _f32),
    )(x, p0, p1, w, b)


def _layer2_call(x, p0, p1, d, w, b):
    return pl.pallas_call(
        _layer2_body,
        grid=(N // _BR,),
        in_specs=[_row_spec(D), _row_spec(D), _row_spec(D), _row_spec(1),
                  _full_spec(D, C), _full_spec(1, C)],
        out_specs=_row_spec(C),
        out_shape=jax.ShapeDtypeStruct((N, C), _f32),
    )(x, p0, p1, d, w, b)


def kernel(features, edge_index, W0, b0, W1, b1, W2, b2):
    src = edge_index[0]
    dst = edge_index[1]
    pairs, counts, offs = _bin_call(src, dst)
    aggp, degp = _maxb_call(features, pairs, counts, offs)
    agg0 = aggp[:N]
    deg = degp[:N].reshape(N, 1)
    h1 = _layer0_call(features, agg0, W0, b0.reshape(1, D))
    src2 = src.reshape(NGCH, GCH)
    dst2 = dst.reshape(NGCH, GCH)
    p = _sum_call(h1, src2, dst2)
    h2 = _layer1_call(h1, p[0, :N], p[1, :N], W1, b1.reshape(1, D))
    p2 = _sum_call(h2, src2, dst2)
    return _layer2_call(h2, p2[0, :N], p2[1, :N], deg, W2, b2.reshape(1, C))


# speedup vs baseline: 1.1591x; 1.0018x over previous
import functools

import jax
import jax.numpy as jnp
from jax import lax
from jax.experimental import pallas as pl
from jax.experimental.pallas import tpu as pltpu
from jax.experimental.pallas import tpu_sc as plsc

_f32 = jnp.float32
_i32 = jnp.int32

N = 10000
E = 320000
D = 128
C = 40

NW = 32
NPW = 320
NPAD = NW * NPW
SCH = 512
NSCH = E // SCH
GCH = 128
NGCH = E // GCH
RPS = 632
NACC = 16 * RPS


def _sc_mesh():
    return plsc.VectorSubcoreMesh(core_axis_name="c", subcore_axis_name="s")


def _onehot0():
    lane = lax.iota(_i32, 16)
    one = jnp.full((16,), 1.0, _f32)
    zero = jnp.full((16,), 0.0, _f32)
    return jnp.where(lane == jnp.zeros((16,), _i32), one, zero)


_SC_PARAMS = pltpu.CompilerParams(needs_layout_passes=False)


EPW = E // NW
BGRP = EPW // 16
PRV = 10304
ROWL = 10752
OMUL = 6554
OSH = 9


def _bin_body(srca, dsta, pairs, counts, offs, srcv, dstv, pairsv, hist,
              offsv, cursv):
    wid = lax.axis_index("c") * 16 + lax.axis_index("s")
    zi = jnp.zeros((16,), _i32)
    omulv = jnp.full((16,), OMUL, _i32)
    npwv = jnp.full((16,), NPW, _i32)
    shov = jnp.full((16,), 21, _i32)
    sh9 = jnp.full((16,), OSH, _i32)
    m511 = jnp.full((16,), 511, _i32)
    onev = jnp.full((16,), 1, _i32)

    pltpu.sync_copy(dsta.at[pl.ds(wid * EPW, EPW)], dstv)
    pltpu.sync_copy(srca.at[pl.ds(wid * EPW, EPW)], srcv)
    for k in range(3):
        hist[pl.ds(k * 16, 16)] = zi

    def h_g(g, _):
        d16 = dstv[pl.ds(g * 16, 16)]
        ow = (d16 * omulv) >> shov
        rank, last = plsc.scan_count(ow)
        plsc.addupdate_scatter(hist, [ow], rank, mask=last)
        return 0

    lax.fori_loop(0, BGRP, h_g, 0)

    sev = jnp.full((16,), 7, _i32)
    m8 = jnp.full((16,), ~7, _i32)
    h0 = hist[pl.ds(0, 16)]
    h1 = hist[pl.ds(16, 16)]
    hp0 = (h0 + sev) & m8
    hp1 = (h1 + sev) & m8
    c0 = plsc.cumsum(hp0)
    c1 = plsc.cumsum(hp1)
    off0 = c0 - hp0
    off1 = c1 - hp1 + jnp.full((16,), c0[15], _i32)
    offsv[pl.ds(0, 16)] = off0
    offsv[pl.ds(16, 16)] = off1
    offsv[pl.ds(32, 16)] = zi
    cursv[pl.ds(0, 16)] = off0
    cursv[pl.ds(16, 16)] = off1

    def p_g(g, _):
        d16 = dstv[pl.ds(g * 16, 16)]
        s16 = srcv[pl.ds(g * 16, 16)]
        ow = (d16 * omulv) >> shov
        dloc = d16 - ow * npwv
        packed = (s16 << sh9) | dloc
        rank, last = plsc.scan_count(ow)
        cur = plsc.load_gather(cursv, [ow])
        pos = cur + rank - onev
        plsc.store_scatter(pairsv, [pos], packed)
        plsc.addupdate_scatter(cursv, [ow], rank, mask=last)
        return 0

    lax.fori_loop(0, BGRP, p_g, 0)

    pltpu.sync_copy(pairsv, pairs.at[pl.ds(wid * ROWL, PRV)])
    pltpu.sync_copy(hist, counts.at[pl.ds(wid * 48, 48)])
    pltpu.sync_copy(offsv, offs.at[pl.ds(wid * 48, 48)])


def _bin_call(src, dst):
    fn = pl.kernel(
        _bin_body,
        out_type=(jax.ShapeDtypeStruct((NW * ROWL,), _i32),
                  jax.ShapeDtypeStruct((NW * 48,), _i32),
                  jax.ShapeDtypeStruct((NW * 48,), _i32)),
        mesh=_sc_mesh(),
        compiler_params=_SC_PARAMS,
        scratch_types=[
            pltpu.VMEM((EPW,), _i32),
            pltpu.VMEM((EPW,), _i32),
            pltpu.VMEM((PRV,), _i32),
            pltpu.VMEM((48,), _i32),
            pltpu.VMEM((48,), _i32),
            pltpu.VMEM((48,), _i32),
        ],
    )
    return fn(src, dst)


def _maxb_body(feat, pairs, counts, offs, out, deg, acc, acc1, degv, cntv,
               offv, pbuf, srcbuf, dlbuf, rowsr, semg):
    wid = lax.axis_index("c") * 16 + lax.axis_index("s")
    lo = wid * NPW
    ninf = jnp.full((16,), -jnp.inf, _f32)
    zf = jnp.zeros((16,), _f32)
    sh9 = jnp.full((16,), OSH, _i32)
    m511 = jnp.full((16,), 511, _i32)
    npwv = jnp.full((16,), NPW, _i32)
    lanev = lax.iota(_i32, 16)
    trash16 = (lanev << sh9) | npwv
    onehot = _onehot0()

    def init_row(r, _):
        for f in range(D // 16):
            acc[r, pl.ds(f * 16, 16)] = ninf
            acc1[r, pl.ds(f * 16, 16)] = ninf
        return 0

    lax.fori_loop(0, NPW + 16, init_row, 0)

    def init_deg(r, _):
        degv[pl.ds(r * 16, 16)] = zf
        return 0

    lax.fori_loop(0, (NPW + 32) // 16, init_deg, 0)

    def initb(q, _):
        pbuf[pl.ds(512 + q * 16, 16)] = trash16
        return 0

    lax.fori_loop(0, 2, initb, 0)

    pltpu.sync_copy(counts, cntv.at[pl.ds(0, NW * 48)])
    pltpu.sync_copy(offs, offv.at[pl.ds(0, NW * 48)])

    def binner(b, _):
        cnt = cntv[pl.ds(b * 48 + wid, 16)][0]
        off = offv[pl.ds(b * 48 + wid, 16)][0]
        nchk = (cnt + 511) // 512

        def chunk(t, _):
            o = pl.multiple_of(b * ROWL + off + t * 512, 8)
            pltpu.sync_copy(pairs.at[pl.ds(o, 512)],
                            pbuf.at[pl.ds(0, 512)])
            valid = jnp.minimum(cnt - t * 512, 512)
            pbuf[pl.ds(valid, 16)] = trash16
            ngrp = (valid + 15) // 16

            @pl.when(ngrp > 0)
            def _():
                def sfill(q, _):
                    sv = pbuf[pl.ds(q * 16, 16)] >> sh9
                    sv = jnp.minimum(jnp.maximum(sv, jnp.zeros((16,), _i32)),
                                     jnp.full((16,), N - 1, _i32))
                    srcbuf[pl.ds(q * 16, 16)] = sv
                    return 0

                lax.fori_loop(0, 33, sfill, 0)
                nblk = (ngrp + 7) // 8
                pltpu.async_copy(feat.at[srcbuf.at[pl.ds(0, 128)]],
                                 rowsr.at[0], semg)

                def blk(k, _):
                    b2 = lax.rem(k, 2)
                    pltpu.make_async_copy(feat.at[srcbuf.at[pl.ds(0, 128)]],
                                          rowsr.at[b2], semg).wait()

                    @pl.when(k + 1 < nblk)
                    def _():
                        pltpu.async_copy(
                            feat.at[srcbuf.at[pl.ds((k + 1) * 128, 128)]],
                            rowsr.at[1 - b2], semg)

                    gcount = jnp.minimum(ngrp - k * 8, 8)

                    def grp(gg, _):
                        g = k * 8 + gg
                        pg = pbuf[pl.ds(g * 16, 16)]
                        dl16 = pg & m511
                        dlbuf[pl.ds(0, 16)] = dl16
                        rankd, lastd = plsc.scan_count(dl16)
                        plsc.addupdate_scatter(degv, [dl16],
                                               rankd.astype(_f32), mask=lastd)

                        rs = []
                        for ee in range(16):
                            rs.append(dlbuf[pl.ds(ee, 16)][0])
                        for ee in range(16):
                            a = acc if ee % 2 == 0 else acc1
                            r = rs[ee]
                            wb = gg * 16 + ee
                            for f in range(D // 16):
                                sl = pl.ds(f * 16, 16)
                                a[r, sl] = jnp.maximum(a[r, sl],
                                                       rowsr[b2, wb, sl])
                        return 0

                    lax.fori_loop(0, gcount, grp, 0)
                    return 0

                lax.fori_loop(0, nblk, blk, 0)

            return 0

        lax.fori_loop(0, nchk, chunk, 0)
        return 0

    lax.fori_loop(0, NW, binner, 0)

    def comb(r, _):
        for f in range(D // 16):
            sl = pl.ds(f * 16, 16)
            acc[r, sl] = jnp.maximum(acc[r, sl], acc1[r, sl])
        return 0

    lax.fori_loop(0, NPW, comb, 0)

    pltpu.sync_copy(acc.at[pl.ds(0, NPW)], out.at[pl.ds(lo, NPW)])
    pltpu.sync_copy(degv.at[pl.ds(0, NPW)], deg.at[pl.ds(lo, NPW)])


def _maxb_call(feat, pairs, counts, offs):
    fn = pl.kernel(
        _maxb_body,
        out_type=(jax.ShapeDtypeStruct((NPAD, D), _f32),
                  jax.ShapeDtypeStruct((NPAD,), _f32)),
        mesh=_sc_mesh(),
        compiler_params=_SC_PARAMS,
        scratch_types=[
            pltpu.VMEM((NPW + 16, D), _f32),
            pltpu.VMEM((NPW + 16, D), _f32),
            pltpu.VMEM((NPW + 32,), _f32),
            pltpu.VMEM((NW * 48 + 16,), _i32),
            pltpu.VMEM((NW * 48 + 16,), _i32),
            pltpu.VMEM((544,), _i32),
            pltpu.VMEM((544,), _i32),
            pltpu.VMEM((32,), _i32),
            pltpu.VMEM((2, 128, D), _f32),
            pltpu.SemaphoreType.DMA,
        ],
    )
    return fn(feat, pairs, counts, offs)


def _max_deg_body(feat, srca, dsta, out, deg, acc, degv, dstc2, srcc2, mdst,
                  msrc, rows2, semd, sems, semg):
    wid = lax.axis_index("c") * 16 + lax.axis_index("s")
    lo = wid * NPW
    ninf = jnp.full((16,), -jnp.inf, _f32)
    zf = jnp.zeros((16,), _f32)
    zi = jnp.zeros((16,), _i32)
    lov = jnp.full((16,), lo, _i32)
    npwv = jnp.full((16,), NPW, _i32)
    onehot = _onehot0()

    def init_row(r, _):
        for f in range(D // 16):
            acc[r, pl.ds(f * 16, 16)] = ninf
        return 0

    lax.fori_loop(0, NPW + 16, init_row, 0)

    def init_deg(r, _):
        degv[pl.ds(r * 16, 16)] = zf
        return 0

    lax.fori_loop(0, (NPW + 32) // 16, init_deg, 0)

    pltpu.async_copy(dsta.at[pl.ds(0, SCH)], dstc2.at[0], semd)
    pltpu.async_copy(srca.at[pl.ds(0, SCH)], srcc2.at[0], sems)

    def process(buf, next_ch):
        pltpu.make_async_copy(dsta.at[pl.ds(0, SCH)], dstc2.at[buf],
                              semd).wait()
        pltpu.make_async_copy(srca.at[pl.ds(0, SCH)], srcc2.at[buf],
                              sems).wait()

        @pl.when(next_ch < NSCH)
        def _():
            nbase = next_ch * SCH
            pltpu.async_copy(dsta.at[pl.ds(nbase, SCH)], dstc2.at[1 - buf],
                             semd)
            pltpu.async_copy(srca.at[pl.ds(nbase, SCH)], srcc2.at[1 - buf],
                             sems)

        def group(g, cnt):
            d16 = dstc2[buf, pl.ds(g * 16, 16)]
            dloc = d16 - lov
            m = (dloc >= zi) & (dloc < npwv)
            s16 = srcc2[buf, pl.ds(g * 16, 16)]
            plsc.store_compressed(mdst.at[pl.ds(cnt, 16)], dloc, mask=m)
            plsc.store_compressed(msrc.at[pl.ds(cnt, 16)], s16, mask=m)
            return cnt + plsc.all_reduce_population_count(m)[0]

        M = lax.fori_loop(0, SCH // 16, group, 0)
        mdst[pl.ds(M, 16)] = npwv
        msrc[pl.ds(M, 16)] = lax.iota(_i32, 16)
        ngrp = (M + 15) // 16

        @pl.when(ngrp > 0)
        def _():
            idx0 = msrc[pl.ds(0, 16)]
            pltpu.async_copy(feat.at[idx0], rows2.at[0], semg)

            def proc(g, _):
                b = lax.rem(g, 2)
                pltpu.make_async_copy(feat.at[idx0], rows2.at[b], semg).wait()

                @pl.when(g + 1 < ngrp)
                def _():
                    idxn = msrc[pl.ds((g + 1) * 16, 16)]
                    pltpu.async_copy(feat.at[idxn], rows2.at[1 - b], semg)

                def edge(e, _):
                    r = mdst[pl.ds(g * 16 + e, 16)][0]
                    for f in range(D // 16):
                        sl = pl.ds(f * 16, 16)
                        acc[r, sl] = jnp.maximum(acc[r, sl], rows2[b, e, sl])
                    dsl = pl.ds(r, 16)
                    degv[dsl] = degv[dsl] + onehot
                    return 0

                lax.fori_loop(0, 16, edge, 0)
                return 0

            lax.fori_loop(0, ngrp, proc, 0)

        return 0

    def pair(j, _):
        process(0, 2 * j + 1)
        process(1, 2 * j + 2)
        return 0

    lax.fori_loop(0, NSCH // 2, pair, 0)
    process(0, jnp.int32(NSCH))

    pltpu.sync_copy(acc.at[pl.ds(0, NPW)], out.at[pl.ds(lo, NPW)])
    pltpu.sync_copy(degv.at[pl.ds(0, NPW)], deg.at[pl.ds(lo, NPW)])


def _max_deg_call(feat, src, dst):
    fn = pl.kernel(
        _max_deg_body,
        out_type=(jax.ShapeDtypeStruct((NPAD, D), _f32),
                  jax.ShapeDtypeStruct((NPAD,), _f32)),
        mesh=_sc_mesh(),
        compiler_params=_SC_PARAMS,
        scratch_types=[
            pltpu.VMEM((NPW + 16, D), _f32),
            pltpu.VMEM((NPW + 32,), _f32),
            pltpu.VMEM((2, SCH), _i32),
            pltpu.VMEM((2, SCH), _i32),
            pltpu.VMEM((SCH + 32,), _i32),
            pltpu.VMEM((SCH + 32,), _i32),
            pltpu.VMEM((2, 16, D), _f32),
            pltpu.SemaphoreType.DMA,
            pltpu.SemaphoreType.DMA,
            pltpu.SemaphoreType.DMA,
        ],
    )
    return fn(feat, src, dst)


def _sum_body(h, src2, dst2, out, acc, srcc2, dstc2, rows2, semis, semid,
              semg, sema):
    c = lax.axis_index("c")
    s = lax.axis_index("s")
    wid = c * 16 + s
    zf = jnp.zeros((16,), _f32)
    cw = (NGCH - wid + NW - 1) // NW

    def zrow(r, _):
        for f in range(D // 16):
            rows2[0, r, pl.ds(f * 16, 16)] = zf
        return 0

    lax.fori_loop(0, GCH, zrow, 0)
    base = s * RPS

    def zacc(t, _):
        pltpu.sync_copy(rows2.at[0], acc.at[pl.ds(base + t * GCH, GCH)])
        return 0

    lax.fori_loop(0, RPS // GCH, zacc, 0)
    tail = RPS - (RPS // GCH) * GCH
    pltpu.sync_copy(rows2.at[0].at[pl.ds(0, tail)],
                    acc.at[pl.ds(base + (RPS // GCH) * GCH, tail)])
    plsc.subcore_barrier()

    def fire_idx(j, b):
        ch = wid + j * NW
        pltpu.async_copy(src2.at[ch], srcc2.at[b], semis)
        pltpu.async_copy(dst2.at[ch], dstc2.at[b], semid)

    def wait_idx(b):
        pltpu.make_async_copy(src2.at[0], srcc2.at[b], semis).wait()
        pltpu.make_async_copy(dst2.at[0], dstc2.at[b], semid).wait()

    def fire_gather(b):
        pltpu.async_copy(h.at[srcc2.at[b]], rows2.at[b], semg)

    def wait_gather(b):
        pltpu.make_async_copy(h.at[srcc2.at[0]], rows2.at[b], semg).wait()

    def fire_scatter(b):
        pltpu.async_copy(rows2.at[b], acc.at[dstc2.at[b]], sema, add=True)

    def wait_scatter():
        pltpu.make_async_copy(rows2.at[0], acc.at[dstc2.at[0]], sema).wait()

    @pl.when(cw > 0)
    def _():
        fire_idx(0, 0)
        wait_idx(0)

        @pl.when(cw > 1)
        def _():
            fire_idx(1, 1)

        fire_gather(0)

    def step(j, _):
        b = lax.rem(j, 3)
        nb = lax.rem(j + 1, 3)
        nnb = lax.rem(j + 2, 3)
        wait_gather(b)
        fire_scatter(b)

        @pl.when(j + 1 < cw)
        def _():
            wait_idx(nb)

            @pl.when(j + 2 < cw)
            def _():
                fire_idx(j + 2, nnb)

            @pl.when(j >= 2)
            def _():
                wait_scatter()

            fire_gather(nb)

        return 0

    lax.fori_loop(0, cw, step, 0)

    @pl.when(cw >= 1)
    def _():
        wait_scatter()

    @pl.when(cw >= 2)
    def _():
        wait_scatter()

    @pl.when(cw >= 3)
    def _():
        wait_scatter()

    plsc.subcore_barrier()
    pltpu.sync_copy(acc.at[pl.ds(s * RPS, RPS)],
                    out.at[c].at[pl.ds(s * RPS, RPS)])


def _sum_call(h, src2, dst2):
    fn = pl.kernel(
        _sum_body,
        out_type=jax.ShapeDtypeStruct((2, NACC, D), _f32),
        mesh=_sc_mesh(),
        compiler_params=_SC_PARAMS,
        scratch_types=[
            pltpu.VMEM_SHARED((NACC, D), _f32),
            pltpu.VMEM((3, GCH), _i32),
            pltpu.VMEM((3, GCH), _i32),
            pltpu.VMEM((3, GCH, D), _f32),
            pltpu.SemaphoreType.DMA,
            pltpu.SemaphoreType.DMA,
            pltpu.SemaphoreType.DMA,
            pltpu.SemaphoreType.DMA,
        ],
    )
    return fn(h, src2, dst2)


_BR = 1000


def _layer0_body(x_ref, a_ref, w_ref, b_ref, o_ref):
    a = a_ref[...]
    agg = jnp.where(jnp.isfinite(a), a, 0.0)
    rst = x_ref[...] + agg
    o_ref[...] = jnp.maximum(
        jnp.dot(rst, w_ref[...], preferred_element_type=_f32) + b_ref[...], 0.0)


def _layer1_body(x_ref, p0_ref, p1_ref, w_ref, b_ref, o_ref):
    rst = x_ref[...] + p0_ref[...] + p1_ref[...]
    o_ref[...] = jnp.maximum(
        jnp.dot(rst, w_ref[...], preferred_element_type=_f32) + b_ref[...], 0.0)


def _layer2_body(x_ref, p0_ref, p1_ref, d_ref, w_ref, b_ref, o_ref):
    dinv = 1.0 / jnp.maximum(d_ref[...], 1.0)
    rst = x_ref[...] + (p0_ref[...] + p1_ref[...]) * dinv
    o_ref[...] = jnp.dot(rst, w_ref[...], preferred_element_type=_f32) + b_ref[...]


def _row_spec(cols):
    return pl.BlockSpec((_BR, cols), lambda i: (i, 0))


def _full_spec(r, c):
    return pl.BlockSpec((r, c), lambda i: (0, 0))


def _layer0_call(x, a, w, b):
    return pl.pallas_call(
        _layer0_body,
        grid=(N // _BR,),
        in_specs=[_row_spec(D), _row_spec(D), _full_spec(D, D), _full_spec(1, D)],
        out_specs=_row_spec(D),
        out_shape=jax.ShapeDtypeStruct((N, D), _f32),
    )(x, a, w, b)


def _layer1_call(x, p0, p1, w, b):
    return pl.pallas_call(
        _layer1_body,
        grid=(N // _BR,),
        in_specs=[_row_spec(D), _row_spec(D), _row_spec(D), _full_spec(D, D),
                  _full_spec(1, D)],
        out_specs=_row_spec(D),
        out_shape=jax.ShapeDtypeStruct((N, D), _f32),
    )(x, p0, p1, w, b)


def _layer2_call(x, p0, p1, d, w, b):
    return pl.pallas_call(
        _layer2_body,
        grid=(N // _BR,),
        in_specs=[_row_spec(D), _row_spec(D), _row_spec(D), _row_spec(1),
                  _full_spec(D, C), _full_spec(1, C)],
        out_specs=_row_spec(C),
        out_shape=jax.ShapeDtypeStruct((N, C), _f32),
    )(x, p0, p1, d, w, b)


def kernel(features, edge_index, W0, b0, W1, b1, W2, b2):
    src = edge_index[0]
    dst = edge_index[1]
    pairs, counts, offs = _bin_call(src, dst)
    aggp, degp = _maxb_call(features, pairs, counts, offs)
    agg0 = aggp[:N]
    deg = degp[:N].reshape(N, 1)
    h1 = _layer0_call(features, agg0, W0, b0.reshape(1, D))
    src2 = src.reshape(NGCH, GCH)
    dst2 = dst.reshape(NGCH, GCH)
    p = _sum_call(h1, src2, dst2)
    h2 = _layer1_call(h1, p[0, :N], p[1, :N], W1, b1.reshape(1, D))
    p2 = _sum_call(h2, src2, dst2)
    return _layer2_call(h2, p2[0, :N], p2[1, :N], deg, W2, b2.reshape(1, C))


# speedup vs baseline: 1.3817x; 1.1920x over previous
import functools

import jax
import jax.numpy as jnp
from jax import lax
from jax.experimental import pallas as pl
from jax.experimental.pallas import tpu as pltpu
from jax.experimental.pallas import tpu_sc as plsc

_f32 = jnp.float32
_i32 = jnp.int32

N = 10000
E = 320000
D = 128
C = 40

NW = 32
NPW = 320
NPAD = NW * NPW
SCH = 512
NSCH = E // SCH
GCH = 128
NGCH = E // GCH
RPS = 632
NACC = 16 * RPS


def _sc_mesh():
    return plsc.VectorSubcoreMesh(core_axis_name="c", subcore_axis_name="s")


def _onehot0():
    lane = lax.iota(_i32, 16)
    one = jnp.full((16,), 1.0, _f32)
    zero = jnp.full((16,), 0.0, _f32)
    return jnp.where(lane == jnp.zeros((16,), _i32), one, zero)


_SC_PARAMS = pltpu.CompilerParams(needs_layout_passes=False)


EPW = E // NW
BGRP = EPW // 16
PRV = 10304
ROWL = 10752
OMUL = 6554
OSH = 9


def _bin_body(srca, dsta, pairs, counts, offs, srcv, dstv, pairsv, hist,
              offsv, cursv):
    wid = lax.axis_index("c") * 16 + lax.axis_index("s")
    zi = jnp.zeros((16,), _i32)
    omulv = jnp.full((16,), OMUL, _i32)
    npwv = jnp.full((16,), NPW, _i32)
    shov = jnp.full((16,), 21, _i32)
    sh9 = jnp.full((16,), OSH, _i32)
    m511 = jnp.full((16,), 511, _i32)
    onev = jnp.full((16,), 1, _i32)

    pltpu.sync_copy(dsta.at[pl.ds(wid * EPW, EPW)], dstv)
    pltpu.sync_copy(srca.at[pl.ds(wid * EPW, EPW)], srcv)
    for k in range(3):
        hist[pl.ds(k * 16, 16)] = zi

    def h_g(g, _):
        d16 = dstv[pl.ds(g * 16, 16)]
        ow = (d16 * omulv) >> shov
        rank, last = plsc.scan_count(ow)
        plsc.addupdate_scatter(hist, [ow], rank, mask=last)
        return 0

    lax.fori_loop(0, BGRP, h_g, 0)

    sev = jnp.full((16,), 7, _i32)
    m8 = jnp.full((16,), ~7, _i32)
    h0 = hist[pl.ds(0, 16)]
    h1 = hist[pl.ds(16, 16)]
    hp0 = (h0 + sev) & m8
    hp1 = (h1 + sev) & m8
    c0 = plsc.cumsum(hp0)
    c1 = plsc.cumsum(hp1)
    off0 = c0 - hp0
    off1 = c1 - hp1 + jnp.full((16,), c0[15], _i32)
    offsv[pl.ds(0, 16)] = off0
    offsv[pl.ds(16, 16)] = off1
    offsv[pl.ds(32, 16)] = zi
    cursv[pl.ds(0, 16)] = off0
    cursv[pl.ds(16, 16)] = off1

    def p_g(g, _):
        d16 = dstv[pl.ds(g * 16, 16)]
        s16 = srcv[pl.ds(g * 16, 16)]
        ow = (d16 * omulv) >> shov
        dloc = d16 - ow * npwv
        packed = (s16 << sh9) | dloc
        rank, last = plsc.scan_count(ow)
        cur = plsc.load_gather(cursv, [ow])
        pos = cur + rank - onev
        plsc.store_scatter(pairsv, [pos], packed)
        plsc.addupdate_scatter(cursv, [ow], rank, mask=last)
        return 0

    lax.fori_loop(0, BGRP, p_g, 0)

    pltpu.sync_copy(pairsv, pairs.at[pl.ds(wid * ROWL, PRV)])
    pltpu.sync_copy(hist, counts.at[pl.ds(wid * 48, 48)])
    pltpu.sync_copy(offsv, offs.at[pl.ds(wid * 48, 48)])


def _bin_call(src, dst):
    fn = pl.kernel(
        _bin_body,
        out_type=(jax.ShapeDtypeStruct((NW * ROWL,), _i32),
                  jax.ShapeDtypeStruct((NW * 48,), _i32),
                  jax.ShapeDtypeStruct((NW * 48,), _i32)),
        mesh=_sc_mesh(),
        compiler_params=_SC_PARAMS,
        scratch_types=[
            pltpu.VMEM((EPW,), _i32),
            pltpu.VMEM((EPW,), _i32),
            pltpu.VMEM((PRV,), _i32),
            pltpu.VMEM((48,), _i32),
            pltpu.VMEM((48,), _i32),
            pltpu.VMEM((48,), _i32),
        ],
    )
    return fn(src, dst)


def _maxb_body(feat, pairs, counts, offs, out, deg, acc, degv, cntv,
               offv, allp, sortp, hist2, offs2, curs2, sb2, dlbuf, rowsr,
               semg):
    wid = lax.axis_index("c") * 16 + lax.axis_index("s")
    lo = wid * NPW
    ninf = jnp.full((16,), -jnp.inf, _f32)
    zf = jnp.zeros((16,), _f32)
    zi = jnp.zeros((16,), _i32)
    sh9 = jnp.full((16,), OSH, _i32)
    m511 = jnp.full((16,), 511, _i32)
    npwv = jnp.full((16,), NPW, _i32)
    nclampv = jnp.full((16,), N - 1, _i32)
    onev = jnp.full((16,), 1, _i32)
    lanev = lax.iota(_i32, 16)
    trash16 = (lanev << sh9) | npwv

    def init_row(r, _):
        for f in range(D // 16):
            acc[r, pl.ds(f * 16, 16)] = ninf
        return 0

    lax.fori_loop(0, NPW + 16, init_row, 0)

    pltpu.sync_copy(counts, cntv.at[pl.ds(0, NW * 48)])
    pltpu.sync_copy(offs, offv.at[pl.ds(0, NW * 48)])

    def binner(b, cur):
        cnt = cntv[pl.ds(b * 48 + wid, 16)][0]
        off = offv[pl.ds(b * 48 + wid, 16)][0]
        nchk = (cnt + 511) // 512

        def chunk(t, _):
            o = pl.multiple_of(b * ROWL + off + t * 512, 8)
            oc = pl.multiple_of(cur + t * 512, 8)
            pltpu.sync_copy(pairs.at[pl.ds(o, 512)], allp.at[pl.ds(oc, 512)])
            return 0

        lax.fori_loop(0, nchk, chunk, 0)
        seg_end = cur + cnt
        allp[pl.ds(seg_end, 16)] = trash16
        return (seg_end + 15) & ~15

    total = lax.fori_loop(0, NW, binner, 0)
    ngp = total // 16

    for q in range((NPW + 32) // 16):
        hist2[pl.ds(q * 16, 16)] = zi

    def h_g(g, _):
        dl16 = allp[pl.ds(g * 16, 16)] & m511
        rank, last = plsc.scan_count(dl16)
        plsc.addupdate_scatter(hist2, [dl16], rank, mask=last)
        return 0

    lax.fori_loop(0, ngp, h_g, 0)

    carry = jnp.int32(0)
    for q in range((NPW + 32) // 16):
        h = hist2[pl.ds(q * 16, 16)]
        cq = plsc.cumsum(h)
        offq = cq - h + jnp.full((16,), carry, _i32)
        offs2[pl.ds(q * 16, 16)] = offq
        curs2[pl.ds(q * 16, 16)] = offq
        carry = carry + cq[15]

    def p_g(g, _):
        p16 = allp[pl.ds(g * 16, 16)]
        dl16 = p16 & m511
        rank, last = plsc.scan_count(dl16)
        cur = plsc.load_gather(curs2, [dl16])
        pos = cur + rank - onev
        plsc.store_scatter(sortp, [pos], p16)
        plsc.addupdate_scatter(curs2, [dl16], rank, mask=last)
        return 0

    lax.fori_loop(0, ngp, p_g, 0)

    for q in range(NPW // 16):
        degv[pl.ds(q * 16, 16)] = hist2[pl.ds(q * 16, 16)].astype(_f32)

    def sfill(k):
        kb = lax.rem(k, 2)

        def sf(q, _):
            sv = sortp[pl.ds(k * 128 + q * 16, 16)] >> sh9
            sv = jnp.minimum(jnp.maximum(sv, zi), nclampv)
            sb2[kb, pl.ds(q * 16, 16)] = sv
            return 0

        lax.fori_loop(0, 8, sf, 0)

    nblk = (ngp + 7) // 8

    @pl.when(nblk > 0)
    def _():
        sfill(jnp.int32(0))
        pltpu.async_copy(feat.at[sb2.at[0]], rowsr.at[0], semg)

        def blk(k, st):
            b2 = lax.rem(k, 2)
            pltpu.make_async_copy(feat.at[sb2.at[0]], rowsr.at[b2],
                                  semg).wait()

            @pl.when(k + 1 < nblk)
            def _():
                sfill(k + 1)
                pltpu.async_copy(feat.at[sb2.at[lax.rem(k + 1, 2)]],
                                 rowsr.at[1 - b2], semg)

            gcount = jnp.minimum(ngp - k * 8, 8)

            def grp(gg, st):
                g = k * 8 + gg
                dl16 = sortp[pl.ds(g * 16, 16)] & m511
                dlbuf[pl.ds(0, 16)] = dl16
                curn, regs = st[0], list(st[1:])
                for ee in range(16):
                    r = dlbuf[pl.ds(ee, 16)][0]
                    wb = gg * 16 + ee
                    row = [rowsr[b2, wb, pl.ds(f * 16, 16)]
                           for f in range(D // 16)]
                    changed = r != curn

                    @pl.when(changed)
                    def _(curn=curn, regs=regs):
                        for f in range(D // 16):
                            acc[curn, pl.ds(f * 16, 16)] = regs[f]

                    keep = jnp.full((16,), changed, jnp.bool_)
                    regs = [jnp.where(keep, row[f], jnp.maximum(regs[f],
                                                                row[f]))
                            for f in range(D // 16)]
                    curn = r
                return (curn, *regs)

            return lax.fori_loop(0, gcount, grp, st)

        st0 = (jnp.int32(NPW),) + tuple(ninf for _ in range(D // 16))
        st = lax.fori_loop(0, nblk, blk, st0)
        curn, regs = st[0], st[1:]
        for f in range(D // 16):
            acc[curn, pl.ds(f * 16, 16)] = regs[f]

    pltpu.sync_copy(acc.at[pl.ds(0, NPW)], out.at[pl.ds(lo, NPW)])
    pltpu.sync_copy(degv.at[pl.ds(0, NPW)], deg.at[pl.ds(lo, NPW)])


def _maxb_call(feat, pairs, counts, offs):
    fn = pl.kernel(
        _maxb_body,
        out_type=(jax.ShapeDtypeStruct((NPAD, D), _f32),
                  jax.ShapeDtypeStruct((NPAD,), _f32)),
        mesh=_sc_mesh(),
        compiler_params=_SC_PARAMS,
        scratch_types=[
            pltpu.VMEM((NPW + 16, D), _f32),
            pltpu.VMEM((NPW + 32,), _f32),
            pltpu.VMEM((NW * 48 + 16,), _i32),
            pltpu.VMEM((NW * 48 + 16,), _i32),
            pltpu.VMEM((17024,), _i32),
            pltpu.VMEM((17024,), _i32),
            pltpu.VMEM((NPW + 32,), _i32),
            pltpu.VMEM((NPW + 32,), _i32),
            pltpu.VMEM((NPW + 32,), _i32),
            pltpu.VMEM((2, 128), _i32),
            pltpu.VMEM((32,), _i32),
            pltpu.VMEM((2, 128, D), _f32),
            pltpu.SemaphoreType.DMA,
        ],
    )
    return fn(feat, pairs, counts, offs)


def _max_deg_body(feat, srca, dsta, out, deg, acc, degv, dstc2, srcc2, mdst,
                  msrc, rows2, semd, sems, semg):
    wid = lax.axis_index("c") * 16 + lax.axis_index("s")
    lo = wid * NPW
    ninf = jnp.full((16,), -jnp.inf, _f32)
    zf = jnp.zeros((16,), _f32)
    zi = jnp.zeros((16,), _i32)
    lov = jnp.full((16,), lo, _i32)
    npwv = jnp.full((16,), NPW, _i32)
    onehot = _onehot0()

    def init_row(r, _):
        for f in range(D // 16):
            acc[r, pl.ds(f * 16, 16)] = ninf
        return 0

    lax.fori_loop(0, NPW + 16, init_row, 0)

    def init_deg(r, _):
        degv[pl.ds(r * 16, 16)] = zf
        return 0

    lax.fori_loop(0, (NPW + 32) // 16, init_deg, 0)

    pltpu.async_copy(dsta.at[pl.ds(0, SCH)], dstc2.at[0], semd)
    pltpu.async_copy(srca.at[pl.ds(0, SCH)], srcc2.at[0], sems)

    def process(buf, next_ch):
        pltpu.make_async_copy(dsta.at[pl.ds(0, SCH)], dstc2.at[buf],
                              semd).wait()
        pltpu.make_async_copy(srca.at[pl.ds(0, SCH)], srcc2.at[buf],
                              sems).wait()

        @pl.when(next_ch < NSCH)
        def _():
            nbase = next_ch * SCH
            pltpu.async_copy(dsta.at[pl.ds(nbase, SCH)], dstc2.at[1 - buf],
                             semd)
            pltpu.async_copy(srca.at[pl.ds(nbase, SCH)], srcc2.at[1 - buf],
                             sems)

        def group(g, cnt):
            d16 = dstc2[buf, pl.ds(g * 16, 16)]
            dloc = d16 - lov
            m = (dloc >= zi) & (dloc < npwv)
            s16 = srcc2[buf, pl.ds(g * 16, 16)]
            plsc.store_compressed(mdst.at[pl.ds(cnt, 16)], dloc, mask=m)
            plsc.store_compressed(msrc.at[pl.ds(cnt, 16)], s16, mask=m)
            return cnt + plsc.all_reduce_population_count(m)[0]

        M = lax.fori_loop(0, SCH // 16, group, 0)
        mdst[pl.ds(M, 16)] = npwv
        msrc[pl.ds(M, 16)] = lax.iota(_i32, 16)
        ngrp = (M + 15) // 16

        @pl.when(ngrp > 0)
        def _():
            idx0 = msrc[pl.ds(0, 16)]
            pltpu.async_copy(feat.at[idx0], rows2.at[0], semg)

            def proc(g, _):
                b = lax.rem(g, 2)
                pltpu.make_async_copy(feat.at[idx0], rows2.at[b], semg).wait()

                @pl.when(g + 1 < ngrp)
                def _():
                    idxn = msrc[pl.ds((g + 1) * 16, 16)]
                    pltpu.async_copy(feat.at[idxn], rows2.at[1 - b], semg)

                def edge(e, _):
                    r = mdst[pl.ds(g * 16 + e, 16)][0]
                    for f in range(D // 16):
                        sl = pl.ds(f * 16, 16)
                        acc[r, sl] = jnp.maximum(acc[r, sl], rows2[b, e, sl])
                    dsl = pl.ds(r, 16)
                    degv[dsl] = degv[dsl] + onehot
                    return 0

                lax.fori_loop(0, 16, edge, 0)
                return 0

            lax.fori_loop(0, ngrp, proc, 0)

        return 0

    def pair(j, _):
        process(0, 2 * j + 1)
        process(1, 2 * j + 2)
        return 0

    lax.fori_loop(0, NSCH // 2, pair, 0)
    process(0, jnp.int32(NSCH))

    pltpu.sync_copy(acc.at[pl.ds(0, NPW)], out.at[pl.ds(lo, NPW)])
    pltpu.sync_copy(degv.at[pl.ds(0, NPW)], deg.at[pl.ds(lo, NPW)])


def _max_deg_call(feat, src, dst):
    fn = pl.kernel(
        _max_deg_body,
        out_type=(jax.ShapeDtypeStruct((NPAD, D), _f32),
                  jax.ShapeDtypeStruct((NPAD,), _f32)),
        mesh=_sc_mesh(),
        compiler_params=_SC_PARAMS,
        scratch_types=[
            pltpu.VMEM((NPW + 16, D), _f32),
            pltpu.VMEM((NPW + 32,), _f32),
            pltpu.VMEM((2, SCH), _i32),
            pltpu.VMEM((2, SCH), _i32),
            pltpu.VMEM((SCH + 32,), _i32),
            pltpu.VMEM((SCH + 32,), _i32),
            pltpu.VMEM((2, 16, D), _f32),
            pltpu.SemaphoreType.DMA,
            pltpu.SemaphoreType.DMA,
            pltpu.SemaphoreType.DMA,
        ],
    )
    return fn(feat, src, dst)


def _sum_body(h, src2, dst2, out, acc, srcc2, dstc2, rows2, semis, semid,
              semg, sema):
    c = lax.axis_index("c")
    s = lax.axis_index("s")
    wid = c * 16 + s
    zf = jnp.zeros((16,), _f32)
    cw = (NGCH - wid + NW - 1) // NW

    def zrow(r, _):
        for f in range(D // 16):
            rows2[0, r, pl.ds(f * 16, 16)] = zf
        return 0

    lax.fori_loop(0, GCH, zrow, 0)
    base = s * RPS

    def zacc(t, _):
        pltpu.sync_copy(rows2.at[0], acc.at[pl.ds(base + t * GCH, GCH)])
        return 0

    lax.fori_loop(0, RPS // GCH, zacc, 0)
    tail = RPS - (RPS // GCH) * GCH
    pltpu.sync_copy(rows2.at[0].at[pl.ds(0, tail)],
                    acc.at[pl.ds(base + (RPS // GCH) * GCH, tail)])
    plsc.subcore_barrier()

    def fire_idx(j, b):
        ch = wid + j * NW
        pltpu.async_copy(src2.at[ch], srcc2.at[b], semis)
        pltpu.async_copy(dst2.at[ch], dstc2.at[b], semid)

    def wait_idx(b):
        pltpu.make_async_copy(src2.at[0], srcc2.at[b], semis).wait()
        pltpu.make_async_copy(dst2.at[0], dstc2.at[b], semid).wait()

    def fire_gather(b):
        pltpu.async_copy(h.at[srcc2.at[b]], rows2.at[b], semg)

    def wait_gather(b):
        pltpu.make_async_copy(h.at[srcc2.at[0]], rows2.at[b], semg).wait()

    def fire_scatter(b):
        pltpu.async_copy(rows2.at[b], acc.at[dstc2.at[b]], sema, add=True)

    def wait_scatter():
        pltpu.make_async_copy(rows2.at[0], acc.at[dstc2.at[0]], sema).wait()

    @pl.when(cw > 0)
    def _():
        fire_idx(0, 0)
        wait_idx(0)

        @pl.when(cw > 1)
        def _():
            fire_idx(1, 1)

        fire_gather(0)

    def step(j, _):
        b = lax.rem(j, 3)
        nb = lax.rem(j + 1, 3)
        nnb = lax.rem(j + 2, 3)
        wait_gather(b)
        fire_scatter(b)

        @pl.when(j + 1 < cw)
        def _():
            wait_idx(nb)

            @pl.when(j + 2 < cw)
            def _():
                fire_idx(j + 2, nnb)

            @pl.when(j >= 2)
            def _():
                wait_scatter()

            fire_gather(nb)

        return 0

    lax.fori_loop(0, cw, step, 0)

    @pl.when(cw >= 1)
    def _():
        wait_scatter()

    @pl.when(cw >= 2)
    def _():
        wait_scatter()

    @pl.when(cw >= 3)
    def _():
        wait_scatter()

    plsc.subcore_barrier()
    pltpu.sync_copy(acc.at[pl.ds(s * RPS, RPS)],
                    out.at[c].at[pl.ds(s * RPS, RPS)])


def _sum_call(h, src2, dst2):
    fn = pl.kernel(
        _sum_body,
        out_type=jax.ShapeDtypeStruct((2, NACC, D), _f32),
        mesh=_sc_mesh(),
        compiler_params=_SC_PARAMS,
        scratch_types=[
            pltpu.VMEM_SHARED((NACC, D), _f32),
            pltpu.VMEM((3, GCH), _i32),
            pltpu.VMEM((3, GCH), _i32),
            pltpu.VMEM((3, GCH, D), _f32),
            pltpu.SemaphoreType.DMA,
            pltpu.SemaphoreType.DMA,
            pltpu.SemaphoreType.DMA,
            pltpu.SemaphoreType.DMA,
        ],
    )
    return fn(h, src2, dst2)


_BR = 1000


def _layer0_body(x_ref, a_ref, w_ref, b_ref, o_ref):
    a = a_ref[...]
    agg = jnp.where(jnp.isfinite(a), a, 0.0)
    rst = x_ref[...] + agg
    o_ref[...] = jnp.maximum(
        jnp.dot(rst, w_ref[...], preferred_element_type=_f32) + b_ref[...], 0.0)


def _layer1_body(x_ref, p0_ref, p1_ref, w_ref, b_ref, o_ref):
    rst = x_ref[...] + p0_ref[...] + p1_ref[...]
    o_ref[...] = jnp.maximum(
        jnp.dot(rst, w_ref[...], preferred_element_type=_f32) + b_ref[...], 0.0)


def _layer2_body(x_ref, p0_ref, p1_ref, d_ref, w_ref, b_ref, o_ref):
    dinv = 1.0 / jnp.maximum(d_ref[...], 1.0)
    rst = x_ref[...] + (p0_ref[...] + p1_ref[...]) * dinv
    o_ref[...] = jnp.dot(rst, w_ref[...], preferred_element_type=_f32) + b_ref[...]


def _row_spec(cols):
    return pl.BlockSpec((_BR, cols), lambda i: (i, 0))


def _full_spec(r, c):
    return pl.BlockSpec((r, c), lambda i: (0, 0))


def _layer0_call(x, a, w, b):
    return pl.pallas_call(
        _layer0_body,
        grid=(N // _BR,),
        in_specs=[_row_spec(D), _row_spec(D), _full_spec(D, D), _full_spec(1, D)],
        out_specs=_row_spec(D),
        out_shape=jax.ShapeDtypeStruct((N, D), _f32),
    )(x, a, w, b)


def _layer1_call(x, p0, p1, w, b):
    return pl.pallas_call(
        _layer1_body,
        grid=(N // _BR,),
        in_specs=[_row_spec(D), _row_spec(D), _row_spec(D), _full_spec(D, D),
                  _full_spec(1, D)],
        out_specs=_row_spec(D),
        out_shape=jax.ShapeDtypeStruct((N, D), _f32),
    )(x, p0, p1, w, b)


def _layer2_call(x, p0, p1, d, w, b):
    return pl.pallas_call(
        _layer2_body,
        grid=(N // _BR,),
        in_specs=[_row_spec(D), _row_spec(D), _row_spec(D), _row_spec(1),
                  _full_spec(D, C), _full_spec(1, C)],
        out_specs=_row_spec(C),
        out_shape=jax.ShapeDtypeStruct((N, C), _f32),
    )(x, p0, p1, d, w, b)


def kernel(features, edge_index, W0, b0, W1, b1, W2, b2):
    src = edge_index[0]
    dst = edge_index[1]
    pairs, counts, offs = _bin_call(src, dst)
    aggp, degp = _maxb_call(features, pairs, counts, offs)
    agg0 = aggp[:N]
    deg = degp[:N].reshape(N, 1)
    h1 = _layer0_call(features, agg0, W0, b0.reshape(1, D))
    src2 = src.reshape(NGCH, GCH)
    dst2 = dst.reshape(NGCH, GCH)
    p = _sum_call(h1, src2, dst2)
    h2 = _layer1_call(h1, p[0, :N], p[1, :N], W1, b1.reshape(1, D))
    p2 = _sum_call(h2, src2, dst2)
    return _layer2_call(h2, p2[0, :N], p2[1, :N], deg, W2, b2.reshape(1, C))
